# Initial kernel scaffold; baseline (speedup 1.0000x reference)
import jax
import jax.numpy as jnp
from jax.experimental import pallas as pl


def kernel(p, s, edge_attr, Wfc, Wfeat, bfeat, attn_a, W1, b1, W2, b2, ln_g, ln_b, edge_index):
    raise NotImplementedError("write your pallas kernel here")


# speedup vs baseline: 62.9778x; 62.9778x over previous
import functools

import jax
import jax.numpy as jnp
from jax import lax
from jax.experimental import pallas as pl
from jax.experimental.pallas import tpu as pltpu
from jax.experimental.pallas import tpu_sc as plsc

N = 10000
E = 320000
IN_DIM = 128
OUT_DIM = 128
H = 8
DH = 16
FEAT = 16
FFN = 512

CH = 128
NCORES = 2
NSUB = 16
NTILES = NCORES * NSUB
NCHUNKS = E // CH
NPAD = 10240
ROWS_PER_TILE = NPAD // NSUB

BN = 2000
BE = 16000


def _node_proj_body(p_ref, wfc_ref, a1_ref, a2_ref, e8_ref,
                    z_ref, asrc_ref, adst_ref):
    z = jnp.dot(p_ref[...], wfc_ref[...], preferred_element_type=jnp.float32)
    z_ref[...] = z
    e8 = e8_ref[...]
    asrc_ref[...] = jnp.dot(z * a1_ref[...], e8,
                            preferred_element_type=jnp.float32)
    adst_ref[...] = jnp.dot(z * a2_ref[...], e8,
                            preferred_element_type=jnp.float32)


def _node_proj(p, wfc_flat, a1, a2, e8):
    grid = N // BN
    return pl.pallas_call(
        _node_proj_body,
        grid=(grid,),
        in_specs=[
            pl.BlockSpec((BN, IN_DIM), lambda i: (i, 0)),
            pl.BlockSpec((IN_DIM, H * DH), lambda i: (0, 0)),
            pl.BlockSpec((1, H * DH), lambda i: (0, 0)),
            pl.BlockSpec((1, H * DH), lambda i: (0, 0)),
            pl.BlockSpec((H * DH, 16), lambda i: (0, 0)),
        ],
        out_specs=[
            pl.BlockSpec((BN, H * DH), lambda i: (i, 0)),
            pl.BlockSpec((BN, 16), lambda i: (i, 0)),
            pl.BlockSpec((BN, 16), lambda i: (i, 0)),
        ],
        out_shape=[
            jax.ShapeDtypeStruct((N, H * DH), jnp.float32),
            jax.ShapeDtypeStruct((N, 16), jnp.float32),
            jax.ShapeDtypeStruct((N, 16), jnp.float32),
        ],
    )(p, wfc_flat, a1, a2, e8)


E8 = E // 8
BE8 = E8 // 5


def _beta_body(ea_ref, wbig_ref, cb_ref, beta_ref):
    beta_ref[...] = (jnp.dot(ea_ref[...], wbig_ref[...],
                             preferred_element_type=jnp.float32)
                     + cb_ref[...])


def _edge_beta(edge_attr, w_big, cb_big):
    ea_r = edge_attr.reshape(E8, 8 * FEAT)
    return pl.pallas_call(
        _beta_body,
        grid=(E8 // BE8,),
        in_specs=[
            pl.BlockSpec((BE8, 8 * FEAT), lambda i: (i, 0)),
            pl.BlockSpec((8 * FEAT, 128), lambda i: (0, 0)),
            pl.BlockSpec((1, 128), lambda i: (0, 0)),
        ],
        out_specs=pl.BlockSpec((BE8, 128), lambda i: (i, 0)),
        out_shape=jax.ShapeDtypeStruct((E8, 128), jnp.float32),
    )(ea_r, w_big, cb_big)


def _sc_edge_body(src_hbm, dst_hbm, beta_hbm, asrc_hbm, adst_hbm, z_hbm,
                  agg_out, den_out,
                  src_v, dst_v, avs, avd, bv, eexp, zr, agg_sh, den_sh,
                  sem_a, sem_d, sem_b, sem_z):
    c = lax.axis_index("c")
    s = lax.axis_index("s")
    wid = c * NSUB + s
    lane = lax.iota(jnp.int32, 16)
    zeros16 = jnp.zeros((16,), jnp.float32)

    def zero_zr(r, carry):
        for q in range(H):
            zr[r, pl.ds(q * DH, 16)] = zeros16
        eexp[r, pl.ds(0, 16)] = zeros16
        return carry
    lax.fori_loop(0, CH, zero_zr, 0)

    base_row = s * ROWS_PER_TILE
    for q in range(ROWS_PER_TILE // CH):
        pltpu.sync_copy(zr, agg_sh.at[pl.ds(base_row + q * CH, CH)])
        pltpu.sync_copy(eexp, den_sh.at[pl.ds(base_row + q * CH, CH)])
    plsc.subcore_barrier()

    nchunks = NCHUNKS // NTILES + jnp.where(wid < NCHUNKS % NTILES, 1, 0)

    def chunk_body(i, carry):
        base = (i * NTILES + wid) * CH
        pltpu.sync_copy(src_hbm.at[pl.ds(base, CH)], src_v)
        pltpu.sync_copy(dst_hbm.at[pl.ds(base, CH)], dst_v)
        cp_b = pltpu.async_copy(
            beta_hbm.at[pl.ds((i * NTILES + wid) * (CH // 8), CH // 8)],
            bv, sem_b)
        cp_a = pltpu.async_copy(asrc_hbm.at[src_v], avs, sem_a)
        cp_d = pltpu.async_copy(adst_hbm.at[dst_v], avd, sem_d)
        cp_z = pltpu.async_copy(z_hbm.at[src_v], zr, sem_z)
        cp_b.wait()
        cp_a.wait()
        cp_d.wait()

        cp_z.wait()

        def edge_body(b, carry2):
            t = (avs[b, pl.ds(0, 16)] + avd[b, pl.ds(0, 16)]
                 + bv[b // 8, pl.ds((b % 8) * 16, 16)])
            t = jnp.maximum(t, t * jnp.float32(0.01))
            ee = jnp.exp(t)
            eexp[b, pl.ds(0, 16)] = jnp.where(lane < 8, ee, 0.0)
            for h in range(H):
                zr[b, pl.ds(h * DH, 16)] = zr[b, pl.ds(h * DH, 16)] * ee[h]
            return carry2
        lax.fori_loop(0, CH, edge_body, 0)

        pltpu.sync_copy(eexp, den_sh.at[dst_v], add=True)
        pltpu.sync_copy(zr, agg_sh.at[dst_v], add=True)
        return carry

    lax.fori_loop(0, nchunks, chunk_body, 0)
    plsc.subcore_barrier()

    pltpu.sync_copy(agg_sh.at[pl.ds(base_row, ROWS_PER_TILE)],
                    agg_out.at[c, pl.ds(base_row, ROWS_PER_TILE)])
    pltpu.sync_copy(den_sh.at[pl.ds(base_row, ROWS_PER_TILE)],
                    den_out.at[c, pl.ds(base_row, ROWS_PER_TILE)])


def _sc_edge(src, dst, beta, asrc, adst, z):
    mesh = plsc.VectorSubcoreMesh(core_axis_name="c", subcore_axis_name="s")
    fn = pl.kernel(
        _sc_edge_body,
        compiler_params=pltpu.CompilerParams(use_tc_tiling_on_sc=False),
        out_type=[
            jax.ShapeDtypeStruct((NCORES, NPAD, H * DH), jnp.float32),
            jax.ShapeDtypeStruct((NCORES, NPAD, 16), jnp.float32),
        ],
        mesh=mesh,
        scratch_types=[
            pltpu.VMEM((CH,), jnp.int32),
            pltpu.VMEM((CH,), jnp.int32),
            pltpu.VMEM((CH, 16), jnp.float32),
            pltpu.VMEM((CH, 16), jnp.float32),
            pltpu.VMEM((CH // 8, 128), jnp.float32),
            pltpu.VMEM((CH, 16), jnp.float32),
            pltpu.VMEM((CH, H * DH), jnp.float32),
            pltpu.VMEM_SHARED((NPAD, H * DH), jnp.float32),
            pltpu.VMEM_SHARED((NPAD, 16), jnp.float32),
            pltpu.SemaphoreType.DMA,
            pltpu.SemaphoreType.DMA,
            pltpu.SemaphoreType.DMA,
            pltpu.SemaphoreType.DMA,
        ],
    )
    return fn(src, dst, beta, asrc, adst, z)


def _post_body(agg_ref, den_ref, s_ref, e8t_ref, lng_ref, lnb_ref,
               w1_ref, b1_ref, w2_ref, b2_ref, out_ref):
    a = agg_ref[0] + agg_ref[1]
    den = den_ref[0] + den_ref[1]
    den = jnp.where(den > 0.0, den, 1.0)
    den128 = jnp.dot(1.0 / den, e8t_ref[...],
                     preferred_element_type=jnp.float32)
    agg = a * den128
    hfeat = jnp.where(agg > 0.0, agg, jnp.exp(jnp.minimum(agg, 0.0)) - 1.0)
    hfeat = hfeat + s_ref[...]
    mu = jnp.mean(hfeat, axis=-1, keepdims=True)
    xm = hfeat - mu
    var = jnp.mean(xm * xm, axis=-1, keepdims=True)
    xn = xm * lax.rsqrt(var + 1e-6) * lng_ref[...] + lnb_ref[...]
    inter = jax.nn.gelu(jnp.dot(xn, w1_ref[...],
                                preferred_element_type=jnp.float32)
                        + b1_ref[...])
    out_ref[...] = (jnp.dot(inter, w2_ref[...],
                            preferred_element_type=jnp.float32)
                    + b2_ref[...] + hfeat)


def _post(aggraw, denraw, s, e8t, lng, lnb, w1, b1, w2, b2):
    grid = N // BN
    return pl.pallas_call(
        _post_body,
        grid=(grid,),
        in_specs=[
            pl.BlockSpec((NCORES, BN, H * DH), lambda i: (0, i, 0)),
            pl.BlockSpec((NCORES, BN, 16), lambda i: (0, i, 0)),
            pl.BlockSpec((BN, OUT_DIM), lambda i: (i, 0)),
            pl.BlockSpec((16, H * DH), lambda i: (0, 0)),
            pl.BlockSpec((1, OUT_DIM), lambda i: (0, 0)),
            pl.BlockSpec((1, OUT_DIM), lambda i: (0, 0)),
            pl.BlockSpec((OUT_DIM, FFN), lambda i: (0, 0)),
            pl.BlockSpec((1, FFN), lambda i: (0, 0)),
            pl.BlockSpec((FFN, OUT_DIM), lambda i: (0, 0)),
            pl.BlockSpec((1, OUT_DIM), lambda i: (0, 0)),
        ],
        out_specs=pl.BlockSpec((BN, OUT_DIM), lambda i: (i, 0)),
        out_shape=jax.ShapeDtypeStruct((N, OUT_DIM), jnp.float32),
    )(aggraw, denraw, s, e8t, lng, lnb, w1, b1, w2, b2)


def kernel(p, s, edge_attr, Wfc, Wfeat, bfeat, attn_a, W1, b1, W2, b2,
           ln_g, ln_b, edge_index):
    src = edge_index[0].astype(jnp.int32)
    dst = edge_index[1].astype(jnp.int32)

    wfc_flat = Wfc.transpose(1, 0, 2).reshape(IN_DIM, H * DH)
    a1 = attn_a[:, :DH].reshape(1, H * DH)
    a2 = attn_a[:, DH:2 * DH].reshape(1, H * DH)
    a3 = attn_a[:, 2 * DH:]
    w_e16 = jnp.pad(jnp.einsum('hfk,hk->fh', Wfeat, a3), ((0, 0), (0, 8)))
    w_big = jnp.kron(jnp.eye(8, dtype=jnp.float32), w_e16)
    cb_big = jnp.tile(jnp.pad(jnp.einsum('hk,hk->h', bfeat, a3), (0, 8)),
                      8).reshape(1, 128)
    e8 = (jnp.arange(H * DH)[:, None] // DH
          == jnp.arange(16)[None, :]).astype(jnp.float32)

    z, asrc, adst = _node_proj(p, wfc_flat, a1, a2, e8)
    beta = _edge_beta(edge_attr, w_big, cb_big)
    aggraw, denraw = _sc_edge(src, dst, beta, asrc, adst, z)
    e16 = (jnp.arange(16)[:, None]
           == jnp.arange(H * DH)[None, :] // DH).astype(jnp.float32)
    return _post(aggraw, denraw, s, e16, ln_g.reshape(1, OUT_DIM),
                 ln_b.reshape(1, OUT_DIM), W1, b1.reshape(1, FFN),
                 W2, b2.reshape(1, OUT_DIM))


# speedup vs baseline: 71.7273x; 1.1389x over previous
import functools

import jax
import jax.numpy as jnp
from jax import lax
from jax.experimental import pallas as pl
from jax.experimental.pallas import tpu as pltpu
from jax.experimental.pallas import tpu_sc as plsc

N = 10000
E = 320000
IN_DIM = 128
OUT_DIM = 128
H = 8
DH = 16
FEAT = 16
FFN = 512

CH = 80
NCORES = 2
NSUB = 16
NTILES = NCORES * NSUB
NCHUNKS = E // CH
NPAD = 10112
ROWS_PER_TILE = NPAD // NSUB

BN = 2000
BE = 16000


def _node_proj_body(p_ref, wfc_ref, a1_ref, a2_ref, e8_ref,
                    z_ref, asrc_ref, adst_ref):
    z = jnp.dot(p_ref[...], wfc_ref[...], preferred_element_type=jnp.float32)
    z_ref[...] = z
    e8 = e8_ref[...]
    asrc_ref[...] = jnp.dot(z * a1_ref[...], e8,
                            preferred_element_type=jnp.float32)
    adst_ref[...] = jnp.dot(z * a2_ref[...], e8,
                            preferred_element_type=jnp.float32)


def _node_proj(p, wfc_flat, a1, a2, e8):
    grid = N // BN
    return pl.pallas_call(
        _node_proj_body,
        grid=(grid,),
        in_specs=[
            pl.BlockSpec((BN, IN_DIM), lambda i: (i, 0)),
            pl.BlockSpec((IN_DIM, H * DH), lambda i: (0, 0)),
            pl.BlockSpec((1, H * DH), lambda i: (0, 0)),
            pl.BlockSpec((1, H * DH), lambda i: (0, 0)),
            pl.BlockSpec((H * DH, 16), lambda i: (0, 0)),
        ],
        out_specs=[
            pl.BlockSpec((BN, H * DH), lambda i: (i, 0)),
            pl.BlockSpec((BN, 16), lambda i: (i, 0)),
            pl.BlockSpec((BN, 16), lambda i: (i, 0)),
        ],
        out_shape=[
            jax.ShapeDtypeStruct((N, H * DH), jnp.float32),
            jax.ShapeDtypeStruct((N, 16), jnp.float32),
            jax.ShapeDtypeStruct((N, 16), jnp.float32),
        ],
    )(p, wfc_flat, a1, a2, e8)


E8 = E // 8
BE8 = E8 // 5


def _beta_body(ea_ref, wbig_ref, cb_ref, beta_ref):
    beta_ref[...] = (jnp.dot(ea_ref[...], wbig_ref[...],
                             preferred_element_type=jnp.float32)
                     + cb_ref[...])


def _edge_beta(edge_attr, w_big, cb_big):
    ea_r = edge_attr.reshape(E8, 8 * FEAT)
    return pl.pallas_call(
        _beta_body,
        grid=(E8 // BE8,),
        in_specs=[
            pl.BlockSpec((BE8, 8 * FEAT), lambda i: (i, 0)),
            pl.BlockSpec((8 * FEAT, 128), lambda i: (0, 0)),
            pl.BlockSpec((1, 128), lambda i: (0, 0)),
        ],
        out_specs=pl.BlockSpec((BE8, 128), lambda i: (i, 0)),
        out_shape=jax.ShapeDtypeStruct((E8, 128), jnp.float32),
    )(ea_r, w_big, cb_big)


def _sc_edge_body(src_hbm, dst_hbm, beta_hbm, asrc_hbm, adst_hbm, z_hbm,
                  agg_out, den_out,
                  src_v0, src_v1, dst_v0, dst_v1, avs0, avs1, avd0, avd1,
                  bv0, bv1, zr0, zr1, ee0, ee1, agg_sh, den_sh,
                  sga0, sga1, sgd0, sgd1, sgb0, sgb1, sgz0, sgz1,
                  ssa0, ssa1, ssd0, ssd1):
    src_v = (src_v0, src_v1)
    dst_v = (dst_v0, dst_v1)
    avs = (avs0, avs1)
    avd = (avd0, avd1)
    bv = (bv0, bv1)
    zr = (zr0, zr1)
    ee = (ee0, ee1)
    sga = (sga0, sga1)
    sgd = (sgd0, sgd1)
    sgb = (sgb0, sgb1)
    sgz = (sgz0, sgz1)
    ssa = (ssa0, ssa1)
    ssd = (ssd0, ssd1)

    c = lax.axis_index("c")
    s = lax.axis_index("s")
    wid = c * NSUB + s
    lane = lax.iota(jnp.int32, 16)
    zeros16 = jnp.zeros((16,), jnp.float32)

    def zero_zr(r, carry):
        for q in range(H):
            zr0[r, pl.ds(q * DH, 16)] = zeros16
        ee0[r, pl.ds(0, 16)] = zeros16
        return carry
    lax.fori_loop(0, CH, zero_zr, 0)

    base_row = s * ROWS_PER_TILE
    for q in range(ROWS_PER_TILE // CH):
        pltpu.sync_copy(zr0, agg_sh.at[pl.ds(base_row + q * CH, CH)])
        pltpu.sync_copy(ee0, den_sh.at[pl.ds(base_row + q * CH, CH)])
    tail = ROWS_PER_TILE % CH
    if tail:
        tbase = base_row + (ROWS_PER_TILE // CH) * CH
        pltpu.sync_copy(zr0.at[pl.ds(0, tail)], agg_sh.at[pl.ds(tbase, tail)])
        pltpu.sync_copy(ee0.at[pl.ds(0, tail)], den_sh.at[pl.ds(tbase, tail)])
    plsc.subcore_barrier()

    nchunks = NCHUNKS // NTILES + jnp.where(wid < NCHUNKS % NTILES, 1, 0)

    def issue(k, par):
        base = (k * NTILES + wid) * CH
        pltpu.sync_copy(src_hbm.at[pl.ds(base, CH)], src_v[par])
        pltpu.sync_copy(dst_hbm.at[pl.ds(base, CH)], dst_v[par])
        pltpu.async_copy(
            beta_hbm.at[pl.ds((k * NTILES + wid) * (CH // 8), CH // 8)],
            bv[par], sgb[par])
        pltpu.async_copy(asrc_hbm.at[src_v[par]], avs[par], sga[par])
        pltpu.async_copy(adst_hbm.at[dst_v[par]], avd[par], sgd[par])
        pltpu.async_copy(z_hbm.at[src_v[par]], zr[par], sgz[par])

    def wait_gathers(par):
        pltpu.make_async_copy(beta_hbm.at[pl.ds(0, CH // 8)],
                              bv[par], sgb[par]).wait()
        pltpu.make_async_copy(asrc_hbm.at[pl.ds(0, CH)],
                              avs[par], sga[par]).wait()
        pltpu.make_async_copy(adst_hbm.at[pl.ds(0, CH)],
                              avd[par], sgd[par]).wait()
        pltpu.make_async_copy(z_hbm.at[pl.ds(0, CH)],
                              zr[par], sgz[par]).wait()

    def wait_scatters(par):
        pltpu.make_async_copy(ee[par], den_sh.at[pl.ds(0, CH)],
                              ssd[par]).wait()
        pltpu.make_async_copy(zr[par], agg_sh.at[pl.ds(0, CH)],
                              ssa[par]).wait()

    def compute(par):
        def edge_body(b, carry2):
            t = (avs[par][b, pl.ds(0, 16)] + avd[par][b, pl.ds(0, 16)]
                 + bv[par][b // 8, pl.ds((b % 8) * 16, 16)])
            t = jnp.maximum(t, t * jnp.float32(0.01))
            eev = jnp.exp(t)
            ee[par][b, pl.ds(0, 16)] = jnp.where(lane < 8, eev, 0.0)
            for h in range(H):
                zr[par][b, pl.ds(h * DH, 16)] = (
                    zr[par][b, pl.ds(h * DH, 16)] * eev[h])
            return carry2
        lax.fori_loop(0, CH, edge_body, 0)

    def scatter(par):
        pltpu.async_copy(ee[par], den_sh.at[dst_v[par]], ssd[par], add=True)
        pltpu.async_copy(zr[par], agg_sh.at[dst_v[par]], ssa[par], add=True)

    issue(0, 0)
    nouter = (NCHUNKS // NTILES + 2) // 2

    def outer(i2, carry):
        for par in (0, 1):
            k = i2 * 2 + par

            @pl.when(k < nchunks)
            def _(k=k, par=par):
                nxt = 1 - par

                @pl.when(k + 1 < nchunks)
                def _():
                    @pl.when(k >= 1)
                    def _():
                        wait_scatters(nxt)
                    issue(k + 1, nxt)
                wait_gathers(par)
                compute(par)
                scatter(par)
        return carry

    lax.fori_loop(0, nouter, outer, 0)
    for par in (0, 1):
        @pl.when(nchunks > par)
        def _(par=par):
            wait_scatters(par)
    plsc.subcore_barrier()

    pltpu.sync_copy(agg_sh.at[pl.ds(base_row, ROWS_PER_TILE)],
                    agg_out.at[c, pl.ds(base_row, ROWS_PER_TILE)])
    pltpu.sync_copy(den_sh.at[pl.ds(base_row, ROWS_PER_TILE)],
                    den_out.at[c, pl.ds(base_row, ROWS_PER_TILE)])


def _sc_edge(src, dst, beta, asrc, adst, z):
    mesh = plsc.VectorSubcoreMesh(core_axis_name="c", subcore_axis_name="s")
    dbuf = lambda *a: [pltpu.VMEM(*a), pltpu.VMEM(*a)]
    fn = pl.kernel(
        _sc_edge_body,
        compiler_params=pltpu.CompilerParams(use_tc_tiling_on_sc=False),
        out_type=[
            jax.ShapeDtypeStruct((NCORES, NPAD, H * DH), jnp.float32),
            jax.ShapeDtypeStruct((NCORES, NPAD, 16), jnp.float32),
        ],
        mesh=mesh,
        scratch_types=(
            dbuf((CH,), jnp.int32)
            + dbuf((CH,), jnp.int32)
            + dbuf((CH, 16), jnp.float32)
            + dbuf((CH, 16), jnp.float32)
            + dbuf((CH // 8, 128), jnp.float32)
            + dbuf((CH, H * DH), jnp.float32)
            + dbuf((CH, 16), jnp.float32)
            + [
                pltpu.VMEM_SHARED((NPAD, H * DH), jnp.float32),
                pltpu.VMEM_SHARED((NPAD, 16), jnp.float32),
            ]
            + [pltpu.SemaphoreType.DMA] * 12
        ),
    )
    return fn(src, dst, beta, asrc, adst, z)


def _post_body(agg_ref, den_ref, s_ref, e8t_ref, lng_ref, lnb_ref,
               w1_ref, b1_ref, w2_ref, b2_ref, out_ref):
    a = agg_ref[0] + agg_ref[1]
    den = den_ref[0] + den_ref[1]
    den = jnp.where(den > 0.0, den, 1.0)
    den128 = jnp.dot(1.0 / den, e8t_ref[...],
                     preferred_element_type=jnp.float32)
    agg = a * den128
    hfeat = jnp.where(agg > 0.0, agg, jnp.exp(jnp.minimum(agg, 0.0)) - 1.0)
    hfeat = hfeat + s_ref[...]
    mu = jnp.mean(hfeat, axis=-1, keepdims=True)
    xm = hfeat - mu
    var = jnp.mean(xm * xm, axis=-1, keepdims=True)
    xn = xm * lax.rsqrt(var + 1e-6) * lng_ref[...] + lnb_ref[...]
    inter = jax.nn.gelu(jnp.dot(xn, w1_ref[...],
                                preferred_element_type=jnp.float32)
                        + b1_ref[...])
    out_ref[...] = (jnp.dot(inter, w2_ref[...],
                            preferred_element_type=jnp.float32)
                    + b2_ref[...] + hfeat)


def _post(aggraw, denraw, s, e8t, lng, lnb, w1, b1, w2, b2):
    grid = N // BN
    return pl.pallas_call(
        _post_body,
        grid=(grid,),
        in_specs=[
            pl.BlockSpec((NCORES, BN, H * DH), lambda i: (0, i, 0)),
            pl.BlockSpec((NCORES, BN, 16), lambda i: (0, i, 0)),
            pl.BlockSpec((BN, OUT_DIM), lambda i: (i, 0)),
            pl.BlockSpec((16, H * DH), lambda i: (0, 0)),
            pl.BlockSpec((1, OUT_DIM), lambda i: (0, 0)),
            pl.BlockSpec((1, OUT_DIM), lambda i: (0, 0)),
            pl.BlockSpec((OUT_DIM, FFN), lambda i: (0, 0)),
            pl.BlockSpec((1, FFN), lambda i: (0, 0)),
            pl.BlockSpec((FFN, OUT_DIM), lambda i: (0, 0)),
            pl.BlockSpec((1, OUT_DIM), lambda i: (0, 0)),
        ],
        out_specs=pl.BlockSpec((BN, OUT_DIM), lambda i: (i, 0)),
        out_shape=jax.ShapeDtypeStruct((N, OUT_DIM), jnp.float32),
    )(aggraw, denraw, s, e8t, lng, lnb, w1, b1, w2, b2)


def kernel(p, s, edge_attr, Wfc, Wfeat, bfeat, attn_a, W1, b1, W2, b2,
           ln_g, ln_b, edge_index):
    src = edge_index[0].astype(jnp.int32)
    dst = edge_index[1].astype(jnp.int32)

    wfc_flat = Wfc.transpose(1, 0, 2).reshape(IN_DIM, H * DH)
    a1 = attn_a[:, :DH].reshape(1, H * DH)
    a2 = attn_a[:, DH:2 * DH].reshape(1, H * DH)
    a3 = attn_a[:, 2 * DH:]
    w_e16 = jnp.pad(jnp.einsum('hfk,hk->fh', Wfeat, a3), ((0, 0), (0, 8)))
    w_big = jnp.kron(jnp.eye(8, dtype=jnp.float32), w_e16)
    cb_big = jnp.tile(jnp.pad(jnp.einsum('hk,hk->h', bfeat, a3), (0, 8)),
                      8).reshape(1, 128)
    e8 = (jnp.arange(H * DH)[:, None] // DH
          == jnp.arange(16)[None, :]).astype(jnp.float32)

    z, asrc, adst = _node_proj(p, wfc_flat, a1, a2, e8)
    beta = _edge_beta(edge_attr, w_big, cb_big)
    aggraw, denraw = _sc_edge(src, dst, beta, asrc, adst, z)
    e16 = (jnp.arange(16)[:, None]
           == jnp.arange(H * DH)[None, :] // DH).astype(jnp.float32)
    return _post(aggraw, denraw, s, e16, ln_g.reshape(1, OUT_DIM),
                 ln_b.reshape(1, OUT_DIM), W1, b1.reshape(1, FFN),
                 W2, b2.reshape(1, OUT_DIM))


# speedup vs baseline: 71.7984x; 1.0010x over previous
import functools

import jax
import jax.numpy as jnp
from jax import lax
from jax.experimental import pallas as pl
from jax.experimental.pallas import tpu as pltpu
from jax.experimental.pallas import tpu_sc as plsc

N = 10000
E = 320000
IN_DIM = 128
OUT_DIM = 128
H = 8
DH = 16
FEAT = 16
FFN = 512

CH = 80
NCORES = 2
NSUB = 16
NTILES = NCORES * NSUB
NCHUNKS = E // CH
NPAD = 10112
ROWS_PER_TILE = NPAD // NSUB

BN = 2000
BE = 16000


def _node_proj_body(p_ref, wfc_ref, a1_ref, a2_ref, e8_ref,
                    z_ref, asrc_ref, adst_ref):
    z = jnp.dot(p_ref[...], wfc_ref[...], preferred_element_type=jnp.float32)
    z_ref[...] = z
    e8 = e8_ref[...]
    asrc_ref[...] = jnp.dot(z * a1_ref[...], e8,
                            preferred_element_type=jnp.float32)
    adst_ref[...] = jnp.dot(z * a2_ref[...], e8,
                            preferred_element_type=jnp.float32)


def _node_proj(p, wfc_flat, a1, a2, e8):
    grid = N // BN
    return pl.pallas_call(
        _node_proj_body,
        grid=(grid,),
        in_specs=[
            pl.BlockSpec((BN, IN_DIM), lambda i: (i, 0)),
            pl.BlockSpec((IN_DIM, H * DH), lambda i: (0, 0)),
            pl.BlockSpec((1, H * DH), lambda i: (0, 0)),
            pl.BlockSpec((1, H * DH), lambda i: (0, 0)),
            pl.BlockSpec((H * DH, 16), lambda i: (0, 0)),
        ],
        out_specs=[
            pl.BlockSpec((BN, H * DH), lambda i: (i, 0)),
            pl.BlockSpec((BN, 16), lambda i: (i, 0)),
            pl.BlockSpec((BN, 16), lambda i: (i, 0)),
        ],
        out_shape=[
            jax.ShapeDtypeStruct((N, H * DH), jnp.float32),
            jax.ShapeDtypeStruct((N, 16), jnp.float32),
            jax.ShapeDtypeStruct((N, 16), jnp.float32),
        ],
    )(p, wfc_flat, a1, a2, e8)


E8 = E // 8
BE8 = E8 // 5


def _beta_body(ea_ref, wbig_ref, cb_ref, beta_ref):
    beta_ref[...] = (jnp.dot(ea_ref[...], wbig_ref[...],
                             preferred_element_type=jnp.float32)
                     + cb_ref[...])


def _edge_beta(edge_attr, w_big, cb_big):
    ea_r = edge_attr.reshape(E8, 8 * FEAT)
    return pl.pallas_call(
        _beta_body,
        grid=(E8 // BE8,),
        in_specs=[
            pl.BlockSpec((BE8, 8 * FEAT), lambda i: (i, 0)),
            pl.BlockSpec((8 * FEAT, 128), lambda i: (0, 0)),
            pl.BlockSpec((1, 128), lambda i: (0, 0)),
        ],
        out_specs=pl.BlockSpec((BE8, 128), lambda i: (i, 0)),
        out_shape=jax.ShapeDtypeStruct((E8, 128), jnp.float32),
    )(ea_r, w_big, cb_big)


SPLIT_G = 25
SPLIT_W = E // SPLIT_G
SPLIT_R = SPLIT_W // 128


def _split_edges_body(ei_ref, src_ref, dst_ref):
    for r in range(SPLIT_R):
        src_ref[0, pl.ds(r, 1), :] = ei_ref[pl.ds(0, 1), pl.ds(r * 128, 128)]
        dst_ref[0, pl.ds(r, 1), :] = ei_ref[pl.ds(1, 1), pl.ds(r * 128, 128)]


def _split_edges(edge_index):
    src_pk, dst_pk = pl.pallas_call(
        _split_edges_body,
        grid=(SPLIT_G,),
        in_specs=[pl.BlockSpec((2, SPLIT_W), lambda i: (0, i))],
        out_specs=[
            pl.BlockSpec((1, SPLIT_R, 128), lambda i: (i, 0, 0)),
            pl.BlockSpec((1, SPLIT_R, 128), lambda i: (i, 0, 0)),
        ],
        out_shape=[
            jax.ShapeDtypeStruct((SPLIT_G, SPLIT_R, 128), jnp.int32),
            jax.ShapeDtypeStruct((SPLIT_G, SPLIT_R, 128), jnp.int32),
        ],
    )(edge_index)
    return src_pk.reshape(E), dst_pk.reshape(E)


def _sc_edge_body(src_hbm, dst_hbm, beta_hbm, asrc_hbm, adst_hbm, z_hbm,
                  agg_out, den_out,
                  src_v0, src_v1, dst_v0, dst_v1, avs0, avs1, avd0, avd1,
                  bv0, bv1, zr0, zr1, ee0, ee1, agg_sh, den_sh,
                  sga0, sga1, sgd0, sgd1, sgb0, sgb1, sgz0, sgz1,
                  ssa0, ssa1, ssd0, ssd1):
    src_v = (src_v0, src_v1)
    dst_v = (dst_v0, dst_v1)
    avs = (avs0, avs1)
    avd = (avd0, avd1)
    bv = (bv0, bv1)
    zr = (zr0, zr1)
    ee = (ee0, ee1)
    sga = (sga0, sga1)
    sgd = (sgd0, sgd1)
    sgb = (sgb0, sgb1)
    sgz = (sgz0, sgz1)
    ssa = (ssa0, ssa1)
    ssd = (ssd0, ssd1)

    c = lax.axis_index("c")
    s = lax.axis_index("s")
    wid = c * NSUB + s
    lane = lax.iota(jnp.int32, 16)
    zeros16 = jnp.zeros((16,), jnp.float32)

    def zero_zr(r, carry):
        for q in range(H):
            zr0[r, pl.ds(q * DH, 16)] = zeros16
        ee0[r, pl.ds(0, 16)] = zeros16
        return carry
    lax.fori_loop(0, CH, zero_zr, 0)

    base_row = s * ROWS_PER_TILE
    for q in range(ROWS_PER_TILE // CH):
        pltpu.sync_copy(zr0, agg_sh.at[pl.ds(base_row + q * CH, CH)])
        pltpu.sync_copy(ee0, den_sh.at[pl.ds(base_row + q * CH, CH)])
    tail = ROWS_PER_TILE % CH
    if tail:
        tbase = base_row + (ROWS_PER_TILE // CH) * CH
        pltpu.sync_copy(zr0.at[pl.ds(0, tail)], agg_sh.at[pl.ds(tbase, tail)])
        pltpu.sync_copy(ee0.at[pl.ds(0, tail)], den_sh.at[pl.ds(tbase, tail)])
    plsc.subcore_barrier()

    nchunks = NCHUNKS // NTILES + jnp.where(wid < NCHUNKS % NTILES, 1, 0)

    def issue(k, par):
        base = (k * NTILES + wid) * CH
        pltpu.sync_copy(src_hbm.at[pl.ds(base, CH)], src_v[par])
        pltpu.sync_copy(dst_hbm.at[pl.ds(base, CH)], dst_v[par])
        pltpu.async_copy(
            beta_hbm.at[pl.ds((k * NTILES + wid) * (CH // 8), CH // 8)],
            bv[par], sgb[par])
        pltpu.async_copy(asrc_hbm.at[src_v[par]], avs[par], sga[par])
        pltpu.async_copy(adst_hbm.at[dst_v[par]], avd[par], sgd[par])
        pltpu.async_copy(z_hbm.at[src_v[par]], zr[par], sgz[par])

    def wait_gathers(par):
        pltpu.make_async_copy(beta_hbm.at[pl.ds(0, CH // 8)],
                              bv[par], sgb[par]).wait()
        pltpu.make_async_copy(asrc_hbm.at[pl.ds(0, CH)],
                              avs[par], sga[par]).wait()
        pltpu.make_async_copy(adst_hbm.at[pl.ds(0, CH)],
                              avd[par], sgd[par]).wait()
        pltpu.make_async_copy(z_hbm.at[pl.ds(0, CH)],
                              zr[par], sgz[par]).wait()

    def wait_scatters(par):
        pltpu.make_async_copy(ee[par], den_sh.at[pl.ds(0, CH)],
                              ssd[par]).wait()
        pltpu.make_async_copy(zr[par], agg_sh.at[pl.ds(0, CH)],
                              ssa[par]).wait()

    def compute(par):
        def edge_body(b, carry2):
            t = (avs[par][b, pl.ds(0, 16)] + avd[par][b, pl.ds(0, 16)]
                 + bv[par][b // 8, pl.ds((b % 8) * 16, 16)])
            t = jnp.maximum(t, t * jnp.float32(0.01))
            eev = jnp.exp(t)
            ee[par][b, pl.ds(0, 16)] = jnp.where(lane < 8, eev, 0.0)
            for h in range(H):
                zr[par][b, pl.ds(h * DH, 16)] = (
                    zr[par][b, pl.ds(h * DH, 16)] * eev[h])
            return carry2
        lax.fori_loop(0, CH, edge_body, 0)

    def scatter(par):
        pltpu.async_copy(ee[par], den_sh.at[dst_v[par]], ssd[par], add=True)
        pltpu.async_copy(zr[par], agg_sh.at[dst_v[par]], ssa[par], add=True)

    issue(0, 0)
    nouter = (NCHUNKS // NTILES + 2) // 2

    def outer(i2, carry):
        for par in (0, 1):
            k = i2 * 2 + par

            @pl.when(k < nchunks)
            def _(k=k, par=par):
                nxt = 1 - par

                @pl.when(k + 1 < nchunks)
                def _():
                    @pl.when(k >= 1)
                    def _():
                        wait_scatters(nxt)
                    issue(k + 1, nxt)
                wait_gathers(par)
                compute(par)
                scatter(par)
        return carry

    lax.fori_loop(0, nouter, outer, 0)
    for par in (0, 1):
        @pl.when(nchunks > par)
        def _(par=par):
            wait_scatters(par)
    plsc.subcore_barrier()

    pltpu.sync_copy(agg_sh.at[pl.ds(base_row, ROWS_PER_TILE)],
                    agg_out.at[c, pl.ds(base_row, ROWS_PER_TILE)])
    pltpu.sync_copy(den_sh.at[pl.ds(base_row, ROWS_PER_TILE)],
                    den_out.at[c, pl.ds(base_row, ROWS_PER_TILE)])


def _sc_edge(src, dst, beta, asrc, adst, z):
    mesh = plsc.VectorSubcoreMesh(core_axis_name="c", subcore_axis_name="s")
    dbuf = lambda *a: [pltpu.VMEM(*a), pltpu.VMEM(*a)]
    fn = pl.kernel(
        _sc_edge_body,
        compiler_params=pltpu.CompilerParams(use_tc_tiling_on_sc=False),
        out_type=[
            jax.ShapeDtypeStruct((NCORES, NPAD, H * DH), jnp.float32),
            jax.ShapeDtypeStruct((NCORES, NPAD, 16), jnp.float32),
        ],
        mesh=mesh,
        scratch_types=(
            dbuf((CH,), jnp.int32)
            + dbuf((CH,), jnp.int32)
            + dbuf((CH, 16), jnp.float32)
            + dbuf((CH, 16), jnp.float32)
            + dbuf((CH // 8, 128), jnp.float32)
            + dbuf((CH, H * DH), jnp.float32)
            + dbuf((CH, 16), jnp.float32)
            + [
                pltpu.VMEM_SHARED((NPAD, H * DH), jnp.float32),
                pltpu.VMEM_SHARED((NPAD, 16), jnp.float32),
            ]
            + [pltpu.SemaphoreType.DMA] * 12
        ),
    )
    return fn(src, dst, beta, asrc, adst, z)


def _post_body(agg_ref, den_ref, s_ref, e8t_ref, lng_ref, lnb_ref,
               w1_ref, b1_ref, w2_ref, b2_ref, out_ref):
    a = agg_ref[0] + agg_ref[1]
    den = den_ref[0] + den_ref[1]
    den = jnp.where(den > 0.0, den, 1.0)
    den128 = jnp.dot(1.0 / den, e8t_ref[...],
                     preferred_element_type=jnp.float32)
    agg = a * den128
    hfeat = jnp.where(agg > 0.0, agg, jnp.exp(jnp.minimum(agg, 0.0)) - 1.0)
    hfeat = hfeat + s_ref[...]
    mu = jnp.mean(hfeat, axis=-1, keepdims=True)
    xm = hfeat - mu
    var = jnp.mean(xm * xm, axis=-1, keepdims=True)
    xn = xm * lax.rsqrt(var + 1e-6) * lng_ref[...] + lnb_ref[...]
    inter = jax.nn.gelu(jnp.dot(xn, w1_ref[...],
                                preferred_element_type=jnp.float32)
                        + b1_ref[...])
    out_ref[...] = (jnp.dot(inter, w2_ref[...],
                            preferred_element_type=jnp.float32)
                    + b2_ref[...] + hfeat)


def _post(aggraw, denraw, s, e8t, lng, lnb, w1, b1, w2, b2):
    grid = N // BN
    return pl.pallas_call(
        _post_body,
        grid=(grid,),
        in_specs=[
            pl.BlockSpec((NCORES, BN, H * DH), lambda i: (0, i, 0)),
            pl.BlockSpec((NCORES, BN, 16), lambda i: (0, i, 0)),
            pl.BlockSpec((BN, OUT_DIM), lambda i: (i, 0)),
            pl.BlockSpec((16, H * DH), lambda i: (0, 0)),
            pl.BlockSpec((1, OUT_DIM), lambda i: (0, 0)),
            pl.BlockSpec((1, OUT_DIM), lambda i: (0, 0)),
            pl.BlockSpec((OUT_DIM, FFN), lambda i: (0, 0)),
            pl.BlockSpec((1, FFN), lambda i: (0, 0)),
            pl.BlockSpec((FFN, OUT_DIM), lambda i: (0, 0)),
            pl.BlockSpec((1, OUT_DIM), lambda i: (0, 0)),
        ],
        out_specs=pl.BlockSpec((BN, OUT_DIM), lambda i: (i, 0)),
        out_shape=jax.ShapeDtypeStruct((N, OUT_DIM), jnp.float32),
    )(aggraw, denraw, s, e8t, lng, lnb, w1, b1, w2, b2)


def kernel(p, s, edge_attr, Wfc, Wfeat, bfeat, attn_a, W1, b1, W2, b2,
           ln_g, ln_b, edge_index):
    src, dst = _split_edges(edge_index.astype(jnp.int32))

    wfc_flat = Wfc.transpose(1, 0, 2).reshape(IN_DIM, H * DH)
    a1 = attn_a[:, :DH].reshape(1, H * DH)
    a2 = attn_a[:, DH:2 * DH].reshape(1, H * DH)
    a3 = attn_a[:, 2 * DH:]
    w_e16 = jnp.pad(jnp.einsum('hfk,hk->fh', Wfeat, a3), ((0, 0), (0, 8)))
    w_big = jnp.kron(jnp.eye(8, dtype=jnp.float32), w_e16)
    cb_big = jnp.tile(jnp.pad(jnp.einsum('hk,hk->h', bfeat, a3), (0, 8)),
                      8).reshape(1, 128)
    e8 = (jnp.arange(H * DH)[:, None] // DH
          == jnp.arange(16)[None, :]).astype(jnp.float32)

    z, asrc, adst = _node_proj(p, wfc_flat, a1, a2, e8)
    beta = _edge_beta(edge_attr, w_big, cb_big)
    aggraw, denraw = _sc_edge(src, dst, beta, asrc, adst, z)
    e16 = (jnp.arange(16)[:, None]
           == jnp.arange(H * DH)[None, :] // DH).astype(jnp.float32)
    return _post(aggraw, denraw, s, e16, ln_g.reshape(1, OUT_DIM),
                 ln_b.reshape(1, OUT_DIM), W1, b1.reshape(1, FFN),
                 W2, b2.reshape(1, OUT_DIM))


# speedup vs baseline: 77.8659x; 1.0845x over previous
import functools

import jax
import jax.numpy as jnp
from jax import lax
from jax.experimental import pallas as pl
from jax.experimental.pallas import tpu as pltpu
from jax.experimental.pallas import tpu_sc as plsc

N = 10000
E = 320000
IN_DIM = 128
OUT_DIM = 128
H = 8
DH = 16
FEAT = 16
FFN = 512

CH = 80
NCORES = 2
NSUB = 16
NTILES = NCORES * NSUB
NCHUNKS = E // CH
NPAD = 10112
ROWS_PER_TILE = NPAD // NSUB

BN = 2000
BE = 16000


def _node_proj_body(p_ref, wfc_ref, a1_ref, a2_ref, e8_ref,
                    z_ref, asrc_ref, adst_ref):
    z = jnp.dot(p_ref[...], wfc_ref[...], preferred_element_type=jnp.float32)
    z_ref[...] = z
    e8 = e8_ref[...]
    asrc_ref[...] = jnp.dot(z * a1_ref[...], e8,
                            preferred_element_type=jnp.float32)
    adst_ref[...] = jnp.dot(z * a2_ref[...], e8,
                            preferred_element_type=jnp.float32)


def _node_proj(p, wfc_flat, a1, a2, e8):
    grid = N // BN
    return pl.pallas_call(
        _node_proj_body,
        grid=(grid,),
        in_specs=[
            pl.BlockSpec((BN, IN_DIM), lambda i: (i, 0)),
            pl.BlockSpec((IN_DIM, H * DH), lambda i: (0, 0)),
            pl.BlockSpec((1, H * DH), lambda i: (0, 0)),
            pl.BlockSpec((1, H * DH), lambda i: (0, 0)),
            pl.BlockSpec((H * DH, 16), lambda i: (0, 0)),
        ],
        out_specs=[
            pl.BlockSpec((BN, H * DH), lambda i: (i, 0)),
            pl.BlockSpec((BN, 16), lambda i: (i, 0)),
            pl.BlockSpec((BN, 16), lambda i: (i, 0)),
        ],
        out_shape=[
            jax.ShapeDtypeStruct((N, H * DH), jnp.float32),
            jax.ShapeDtypeStruct((N, 16), jnp.float32),
            jax.ShapeDtypeStruct((N, 16), jnp.float32),
        ],
    )(p, wfc_flat, a1, a2, e8)


E8 = E // 8
BE8 = E8 // 5


def _beta_body(ea_ref, wbig_ref, cb_ref, beta_ref):
    beta_ref[...] = (jnp.dot(ea_ref[...], wbig_ref[...],
                             preferred_element_type=jnp.float32)
                     + cb_ref[...])


def _edge_beta(edge_attr, w_big, cb_big):
    ea_r = edge_attr.reshape(E8, 8 * FEAT)
    return pl.pallas_call(
        _beta_body,
        grid=(E8 // BE8,),
        in_specs=[
            pl.BlockSpec((BE8, 8 * FEAT), lambda i: (i, 0)),
            pl.BlockSpec((8 * FEAT, 128), lambda i: (0, 0)),
            pl.BlockSpec((1, 128), lambda i: (0, 0)),
        ],
        out_specs=pl.BlockSpec((BE8, 128), lambda i: (i, 0)),
        out_shape=jax.ShapeDtypeStruct((E8, 128), jnp.float32),
    )(ea_r, w_big, cb_big)


SPLIT_G = 25
SPLIT_W = E // SPLIT_G
SPLIT_R = SPLIT_W // 128


def _split_edges_body(ei_ref, src_ref, dst_ref):
    for r in range(SPLIT_R):
        src_ref[0, pl.ds(r, 1), :] = ei_ref[pl.ds(0, 1), pl.ds(r * 128, 128)]
        dst_ref[0, pl.ds(r, 1), :] = ei_ref[pl.ds(1, 1), pl.ds(r * 128, 128)]


def _split_edges(edge_index):
    src_pk, dst_pk = pl.pallas_call(
        _split_edges_body,
        grid=(SPLIT_G,),
        in_specs=[pl.BlockSpec((2, SPLIT_W), lambda i: (0, i))],
        out_specs=[
            pl.BlockSpec((1, SPLIT_R, 128), lambda i: (i, 0, 0)),
            pl.BlockSpec((1, SPLIT_R, 128), lambda i: (i, 0, 0)),
        ],
        out_shape=[
            jax.ShapeDtypeStruct((SPLIT_G, SPLIT_R, 128), jnp.int32),
            jax.ShapeDtypeStruct((SPLIT_G, SPLIT_R, 128), jnp.int32),
        ],
    )(edge_index)
    return src_pk.reshape(E), dst_pk.reshape(E)


def _sc_edge_body(src_hbm, dst_hbm, beta_hbm, asrc_hbm, adst_hbm, z_hbm,
                  agg_out, den_out,
                  src_v0, src_v1, dst_v0, dst_v1, avs0, avs1, avd0, avd1,
                  bv0, bv1, zr0, zr1, ee0, ee1, agg_sh, den_sh,
                  sga0, sga1, sgd0, sgd1, sgb0, sgb1, sgz0, sgz1,
                  ssa0, ssa1, ssd0, ssd1, sis0, sis1, sid0, sid1):
    src_v = (src_v0, src_v1)
    dst_v = (dst_v0, dst_v1)
    avs = (avs0, avs1)
    avd = (avd0, avd1)
    bv = (bv0, bv1)
    zr = (zr0, zr1)
    ee = (ee0, ee1)
    sga = (sga0, sga1)
    sgd = (sgd0, sgd1)
    sgb = (sgb0, sgb1)
    sgz = (sgz0, sgz1)
    ssa = (ssa0, ssa1)
    ssd = (ssd0, ssd1)
    sis = (sis0, sis1)
    sid = (sid0, sid1)

    c = lax.axis_index("c")
    s = lax.axis_index("s")
    wid = c * NSUB + s
    lane = lax.iota(jnp.int32, 16)
    zeros16 = jnp.zeros((16,), jnp.float32)

    def zero_zr(r, carry):
        for q in range(H):
            zr0[r, pl.ds(q * DH, 16)] = zeros16
        ee0[r, pl.ds(0, 16)] = zeros16
        return carry
    lax.fori_loop(0, CH, zero_zr, 0)

    base_row = s * ROWS_PER_TILE
    for q in range(ROWS_PER_TILE // CH):
        pltpu.sync_copy(zr0, agg_sh.at[pl.ds(base_row + q * CH, CH)])
        pltpu.sync_copy(ee0, den_sh.at[pl.ds(base_row + q * CH, CH)])
    tail = ROWS_PER_TILE % CH
    if tail:
        tbase = base_row + (ROWS_PER_TILE // CH) * CH
        pltpu.sync_copy(zr0.at[pl.ds(0, tail)], agg_sh.at[pl.ds(tbase, tail)])
        pltpu.sync_copy(ee0.at[pl.ds(0, tail)], den_sh.at[pl.ds(tbase, tail)])
    plsc.subcore_barrier()

    nchunks = NCHUNKS // NTILES + jnp.where(wid < NCHUNKS % NTILES, 1, 0)

    def issue(k, par):
        base = (k * NTILES + wid) * CH
        pltpu.async_copy(src_hbm.at[pl.ds(base, CH)], src_v[par], sis[par])
        pltpu.async_copy(dst_hbm.at[pl.ds(base, CH)], dst_v[par], sid[par])
        pltpu.async_copy(
            beta_hbm.at[pl.ds((k * NTILES + wid) * (CH // 8), CH // 8)],
            bv[par], sgb[par])
        pltpu.make_async_copy(src_hbm.at[pl.ds(0, CH)], src_v[par],
                              sis[par]).wait()
        pltpu.async_copy(asrc_hbm.at[src_v[par]], avs[par], sga[par])
        pltpu.async_copy(z_hbm.at[src_v[par]], zr[par], sgz[par])
        pltpu.make_async_copy(dst_hbm.at[pl.ds(0, CH)], dst_v[par],
                              sid[par]).wait()
        pltpu.async_copy(adst_hbm.at[dst_v[par]], avd[par], sgd[par])

    def wait_gathers(par):
        pltpu.make_async_copy(beta_hbm.at[pl.ds(0, CH // 8)],
                              bv[par], sgb[par]).wait()
        pltpu.make_async_copy(asrc_hbm.at[pl.ds(0, CH)],
                              avs[par], sga[par]).wait()
        pltpu.make_async_copy(adst_hbm.at[pl.ds(0, CH)],
                              avd[par], sgd[par]).wait()
        pltpu.make_async_copy(z_hbm.at[pl.ds(0, CH)],
                              zr[par], sgz[par]).wait()

    def wait_scatters(par):
        pltpu.make_async_copy(ee[par], den_sh.at[pl.ds(0, CH)],
                              ssd[par]).wait()
        pltpu.make_async_copy(zr[par], agg_sh.at[pl.ds(0, CH)],
                              ssa[par]).wait()

    def compute(par):
        def edge_body(b, carry2):
            t = (avs[par][b, pl.ds(0, 16)] + avd[par][b, pl.ds(0, 16)]
                 + bv[par][b // 8, pl.ds((b % 8) * 16, 16)])
            t = jnp.maximum(t, t * jnp.float32(0.01))
            eev = jnp.exp(t)
            ee[par][b, pl.ds(0, 16)] = jnp.where(lane < 8, eev, 0.0)
            for h in range(H):
                zr[par][b, pl.ds(h * DH, 16)] = (
                    zr[par][b, pl.ds(h * DH, 16)] * eev[h])
            return carry2
        lax.fori_loop(0, CH, edge_body, 0)

    def scatter(par):
        pltpu.async_copy(ee[par], den_sh.at[dst_v[par]], ssd[par], add=True)
        pltpu.async_copy(zr[par], agg_sh.at[dst_v[par]], ssa[par], add=True)

    issue(0, 0)
    nouter = (NCHUNKS // NTILES + 2) // 2

    def outer(i2, carry):
        for par in (0, 1):
            k = i2 * 2 + par

            @pl.when(k < nchunks)
            def _(k=k, par=par):
                nxt = 1 - par

                @pl.when(k + 1 < nchunks)
                def _():
                    @pl.when(k >= 1)
                    def _():
                        wait_scatters(nxt)
                    issue(k + 1, nxt)
                wait_gathers(par)
                compute(par)
                scatter(par)
        return carry

    lax.fori_loop(0, nouter, outer, 0)
    for par in (0, 1):
        @pl.when(nchunks > par)
        def _(par=par):
            wait_scatters(par)
    plsc.subcore_barrier()

    pltpu.sync_copy(agg_sh.at[pl.ds(base_row, ROWS_PER_TILE)],
                    agg_out.at[c, pl.ds(base_row, ROWS_PER_TILE)])
    pltpu.sync_copy(den_sh.at[pl.ds(base_row, ROWS_PER_TILE)],
                    den_out.at[c, pl.ds(base_row, ROWS_PER_TILE)])


def _sc_edge(src, dst, beta, asrc, adst, z):
    mesh = plsc.VectorSubcoreMesh(core_axis_name="c", subcore_axis_name="s")
    dbuf = lambda *a: [pltpu.VMEM(*a), pltpu.VMEM(*a)]
    fn = pl.kernel(
        _sc_edge_body,
        compiler_params=pltpu.CompilerParams(use_tc_tiling_on_sc=False),
        out_type=[
            jax.ShapeDtypeStruct((NCORES, NPAD, H * DH), jnp.float32),
            jax.ShapeDtypeStruct((NCORES, NPAD, 16), jnp.float32),
        ],
        mesh=mesh,
        scratch_types=(
            dbuf((CH,), jnp.int32)
            + dbuf((CH,), jnp.int32)
            + dbuf((CH, 16), jnp.float32)
            + dbuf((CH, 16), jnp.float32)
            + dbuf((CH // 8, 128), jnp.float32)
            + dbuf((CH, H * DH), jnp.float32)
            + dbuf((CH, 16), jnp.float32)
            + [
                pltpu.VMEM_SHARED((NPAD, H * DH), jnp.float32),
                pltpu.VMEM_SHARED((NPAD, 16), jnp.float32),
            ]
            + [pltpu.SemaphoreType.DMA] * 16
        ),
    )
    return fn(src, dst, beta, asrc, adst, z)


def _post_body(agg_ref, den_ref, s_ref, e8t_ref, lng_ref, lnb_ref,
               w1_ref, b1_ref, w2_ref, b2_ref, out_ref):
    a = agg_ref[0] + agg_ref[1]
    den = den_ref[0] + den_ref[1]
    den = jnp.where(den > 0.0, den, 1.0)
    den128 = jnp.dot(1.0 / den, e8t_ref[...],
                     preferred_element_type=jnp.float32)
    agg = a * den128
    hfeat = jnp.where(agg > 0.0, agg, jnp.exp(jnp.minimum(agg, 0.0)) - 1.0)
    hfeat = hfeat + s_ref[...]
    mu = jnp.mean(hfeat, axis=-1, keepdims=True)
    xm = hfeat - mu
    var = jnp.mean(xm * xm, axis=-1, keepdims=True)
    xn = xm * lax.rsqrt(var + 1e-6) * lng_ref[...] + lnb_ref[...]
    inter = jax.nn.gelu(jnp.dot(xn, w1_ref[...],
                                preferred_element_type=jnp.float32)
                        + b1_ref[...])
    out_ref[...] = (jnp.dot(inter, w2_ref[...],
                            preferred_element_type=jnp.float32)
                    + b2_ref[...] + hfeat)


def _post(aggraw, denraw, s, e8t, lng, lnb, w1, b1, w2, b2):
    grid = N // BN
    return pl.pallas_call(
        _post_body,
        grid=(grid,),
        in_specs=[
            pl.BlockSpec((NCORES, BN, H * DH), lambda i: (0, i, 0)),
            pl.BlockSpec((NCORES, BN, 16), lambda i: (0, i, 0)),
            pl.BlockSpec((BN, OUT_DIM), lambda i: (i, 0)),
            pl.BlockSpec((16, H * DH), lambda i: (0, 0)),
            pl.BlockSpec((1, OUT_DIM), lambda i: (0, 0)),
            pl.BlockSpec((1, OUT_DIM), lambda i: (0, 0)),
            pl.BlockSpec((OUT_DIM, FFN), lambda i: (0, 0)),
            pl.BlockSpec((1, FFN), lambda i: (0, 0)),
            pl.BlockSpec((FFN, OUT_DIM), lambda i: (0, 0)),
            pl.BlockSpec((1, OUT_DIM), lambda i: (0, 0)),
        ],
        out_specs=pl.BlockSpec((BN, OUT_DIM), lambda i: (i, 0)),
        out_shape=jax.ShapeDtypeStruct((N, OUT_DIM), jnp.float32),
    )(aggraw, denraw, s, e8t, lng, lnb, w1, b1, w2, b2)


def kernel(p, s, edge_attr, Wfc, Wfeat, bfeat, attn_a, W1, b1, W2, b2,
           ln_g, ln_b, edge_index):
    src, dst = _split_edges(edge_index.astype(jnp.int32))

    wfc_flat = Wfc.transpose(1, 0, 2).reshape(IN_DIM, H * DH)
    a1 = attn_a[:, :DH].reshape(1, H * DH)
    a2 = attn_a[:, DH:2 * DH].reshape(1, H * DH)
    a3 = attn_a[:, 2 * DH:]
    w_e16 = jnp.pad(jnp.einsum('hfk,hk->fh', Wfeat, a3), ((0, 0), (0, 8)))
    w_big = jnp.kron(jnp.eye(8, dtype=jnp.float32), w_e16)
    cb_big = jnp.tile(jnp.pad(jnp.einsum('hk,hk->h', bfeat, a3), (0, 8)),
                      8).reshape(1, 128)
    e8 = (jnp.arange(H * DH)[:, None] // DH
          == jnp.arange(16)[None, :]).astype(jnp.float32)

    z, asrc, adst = _node_proj(p, wfc_flat, a1, a2, e8)
    beta = _edge_beta(edge_attr, w_big, cb_big)
    aggraw, denraw = _sc_edge(src, dst, beta, asrc, adst, z)
    e16 = (jnp.arange(16)[:, None]
           == jnp.arange(H * DH)[None, :] // DH).astype(jnp.float32)
    return _post(aggraw, denraw, s, e16, ln_g.reshape(1, OUT_DIM),
                 ln_b.reshape(1, OUT_DIM), W1, b1.reshape(1, FFN),
                 W2, b2.reshape(1, OUT_DIM))


# speedup vs baseline: 81.1339x; 1.0420x over previous
import functools

import jax
import jax.numpy as jnp
from jax import lax
from jax.experimental import pallas as pl
from jax.experimental.pallas import tpu as pltpu
from jax.experimental.pallas import tpu_sc as plsc

N = 10000
E = 320000
IN_DIM = 128
OUT_DIM = 128
H = 8
DH = 16
FEAT = 16
FFN = 512

CH = 80
NCORES = 2
NSUB = 16
NTILES = NCORES * NSUB
NCHUNKS = E // CH
NPAD = 10112
ROWS_PER_TILE = NPAD // NSUB

BN = 2000
BE = 16000


def _node_proj_body(p_ref, wfc_ref, a1_ref, a2_ref, e8_ref,
                    z_ref, adst_ref):
    z = jnp.dot(p_ref[...], wfc_ref[...], preferred_element_type=jnp.float32)
    e8 = e8_ref[...]
    z_ref[:, pl.ds(0, H * DH)] = z
    z_ref[:, pl.ds(H * DH, 16)] = jnp.dot(z * a1_ref[...], e8,
                                          preferred_element_type=jnp.float32)
    adst_ref[...] = jnp.dot(z * a2_ref[...], e8,
                            preferred_element_type=jnp.float32)


def _node_proj(p, wfc_flat, a1, a2, e8):
    grid = N // BN
    return pl.pallas_call(
        _node_proj_body,
        grid=(grid,),
        in_specs=[
            pl.BlockSpec((BN, IN_DIM), lambda i: (i, 0)),
            pl.BlockSpec((IN_DIM, H * DH), lambda i: (0, 0)),
            pl.BlockSpec((1, H * DH), lambda i: (0, 0)),
            pl.BlockSpec((1, H * DH), lambda i: (0, 0)),
            pl.BlockSpec((H * DH, 16), lambda i: (0, 0)),
        ],
        out_specs=[
            pl.BlockSpec((BN, H * DH + 16), lambda i: (i, 0)),
            pl.BlockSpec((BN, 16), lambda i: (i, 0)),
        ],
        out_shape=[
            jax.ShapeDtypeStruct((N, H * DH + 16), jnp.float32),
            jax.ShapeDtypeStruct((N, 16), jnp.float32),
        ],
    )(p, wfc_flat, a1, a2, e8)


E8 = E // 8
BE8 = E8 // 5


def _beta_body(ea_ref, wbig_ref, cb_ref, beta_ref):
    beta_ref[...] = (jnp.dot(ea_ref[...], wbig_ref[...],
                             preferred_element_type=jnp.float32)
                     + cb_ref[...])


def _edge_beta(edge_attr, w_big, cb_big):
    ea_r = edge_attr.reshape(E8, 8 * FEAT)
    return pl.pallas_call(
        _beta_body,
        grid=(E8 // BE8,),
        in_specs=[
            pl.BlockSpec((BE8, 8 * FEAT), lambda i: (i, 0)),
            pl.BlockSpec((8 * FEAT, 128), lambda i: (0, 0)),
            pl.BlockSpec((1, 128), lambda i: (0, 0)),
        ],
        out_specs=pl.BlockSpec((BE8, 128), lambda i: (i, 0)),
        out_shape=jax.ShapeDtypeStruct((E8, 128), jnp.float32),
    )(ea_r, w_big, cb_big)


SPLIT_G = 25
SPLIT_W = E // SPLIT_G
SPLIT_R = SPLIT_W // 128


def _split_edges_body(ei_ref, src_ref, dst_ref):
    for r in range(SPLIT_R):
        src_ref[0, pl.ds(r, 1), :] = ei_ref[pl.ds(0, 1), pl.ds(r * 128, 128)]
        dst_ref[0, pl.ds(r, 1), :] = ei_ref[pl.ds(1, 1), pl.ds(r * 128, 128)]


def _split_edges(edge_index):
    src_pk, dst_pk = pl.pallas_call(
        _split_edges_body,
        grid=(SPLIT_G,),
        in_specs=[pl.BlockSpec((2, SPLIT_W), lambda i: (0, i))],
        out_specs=[
            pl.BlockSpec((1, SPLIT_R, 128), lambda i: (i, 0, 0)),
            pl.BlockSpec((1, SPLIT_R, 128), lambda i: (i, 0, 0)),
        ],
        out_shape=[
            jax.ShapeDtypeStruct((SPLIT_G, SPLIT_R, 128), jnp.int32),
            jax.ShapeDtypeStruct((SPLIT_G, SPLIT_R, 128), jnp.int32),
        ],
    )(edge_index)
    return src_pk.reshape(E), dst_pk.reshape(E)


def _sc_edge_body(src_hbm, dst_hbm, beta_hbm, adst_hbm, z_hbm,
                  agg_out, den_out,
                  src_v0, src_v1, dst_v0, dst_v1, avd0, avd1,
                  bv0, bv1, zr0, zr1, agg_sh,
                  sgd0, sgd1, sgb0, sgb1, sgz0, sgz1,
                  ssa0, ssa1, sis0, sis1, sid0, sid1):
    src_v = (src_v0, src_v1)
    dst_v = (dst_v0, dst_v1)
    avd = (avd0, avd1)
    bv = (bv0, bv1)
    zr = (zr0, zr1)
    sgd = (sgd0, sgd1)
    sgb = (sgb0, sgb1)
    sgz = (sgz0, sgz1)
    ssa = (ssa0, ssa1)
    sis = (sis0, sis1)
    sid = (sid0, sid1)

    c = lax.axis_index("c")
    s = lax.axis_index("s")
    wid = c * NSUB + s
    lane = lax.iota(jnp.int32, 16)
    zeros16 = jnp.zeros((16,), jnp.float32)

    def zero_zr(r, carry):
        for q in range(9):
            zr0[r, pl.ds(q * DH, 16)] = zeros16
        return carry
    lax.fori_loop(0, CH, zero_zr, 0)

    base_row = s * ROWS_PER_TILE
    for q in range(ROWS_PER_TILE // CH):
        pltpu.sync_copy(zr0, agg_sh.at[pl.ds(base_row + q * CH, CH)])
    tail = ROWS_PER_TILE % CH
    if tail:
        tbase = base_row + (ROWS_PER_TILE // CH) * CH
        pltpu.sync_copy(zr0.at[pl.ds(0, tail)], agg_sh.at[pl.ds(tbase, tail)])
    plsc.subcore_barrier()

    nchunks = NCHUNKS // NTILES + jnp.where(wid < NCHUNKS % NTILES, 1, 0)

    def issue(k, par):
        base = (k * NTILES + wid) * CH
        pltpu.async_copy(src_hbm.at[pl.ds(base, CH)], src_v[par], sis[par])
        pltpu.async_copy(dst_hbm.at[pl.ds(base, CH)], dst_v[par], sid[par])
        pltpu.async_copy(
            beta_hbm.at[pl.ds((k * NTILES + wid) * (CH // 8), CH // 8)],
            bv[par], sgb[par])
        pltpu.make_async_copy(src_hbm.at[pl.ds(0, CH)], src_v[par],
                              sis[par]).wait()
        pltpu.async_copy(z_hbm.at[src_v[par]], zr[par], sgz[par])
        pltpu.make_async_copy(dst_hbm.at[pl.ds(0, CH)], dst_v[par],
                              sid[par]).wait()
        pltpu.async_copy(adst_hbm.at[dst_v[par]], avd[par], sgd[par])

    def wait_gathers(par):
        pltpu.make_async_copy(beta_hbm.at[pl.ds(0, CH // 8)],
                              bv[par], sgb[par]).wait()
        pltpu.make_async_copy(adst_hbm.at[pl.ds(0, CH)],
                              avd[par], sgd[par]).wait()
        pltpu.make_async_copy(z_hbm.at[pl.ds(0, CH)],
                              zr[par], sgz[par]).wait()

    def wait_scatters(par):
        pltpu.make_async_copy(zr[par], agg_sh.at[pl.ds(0, CH)],
                              ssa[par]).wait()

    def compute(par):
        def edge_body(b, carry2):
            t = (zr[par][b, pl.ds(H * DH, 16)] + avd[par][b, pl.ds(0, 16)]
                 + bv[par][b // 8, pl.ds((b % 8) * 16, 16)])
            t = jnp.maximum(t, t * jnp.float32(0.01))
            eev = jnp.exp(t)
            zr[par][b, pl.ds(H * DH, 16)] = jnp.where(lane < 8, eev, 0.0)
            for h in range(H):
                zr[par][b, pl.ds(h * DH, 16)] = (
                    zr[par][b, pl.ds(h * DH, 16)] * eev[h])
            return carry2
        lax.fori_loop(0, CH, edge_body, 0)

    def scatter(par):
        pltpu.async_copy(zr[par], agg_sh.at[dst_v[par]], ssa[par], add=True)

    issue(0, 0)
    nouter = (NCHUNKS // NTILES + 2) // 2

    def outer(i2, carry):
        for par in (0, 1):
            k = i2 * 2 + par

            @pl.when(k < nchunks)
            def _(k=k, par=par):
                nxt = 1 - par

                @pl.when(k + 1 < nchunks)
                def _():
                    @pl.when(k >= 1)
                    def _():
                        wait_scatters(nxt)
                    issue(k + 1, nxt)
                wait_gathers(par)
                compute(par)
                scatter(par)
        return carry

    lax.fori_loop(0, nouter, outer, 0)
    for par in (0, 1):
        @pl.when(nchunks > par)
        def _(par=par):
            wait_scatters(par)
    plsc.subcore_barrier()

    pltpu.sync_copy(
        agg_sh.at[pl.ds(base_row, ROWS_PER_TILE), pl.ds(0, H * DH)],
        agg_out.at[c, pl.ds(base_row, ROWS_PER_TILE)])
    pltpu.sync_copy(
        agg_sh.at[pl.ds(base_row, ROWS_PER_TILE), pl.ds(H * DH, 16)],
        den_out.at[c, pl.ds(base_row, ROWS_PER_TILE)])


def _sc_edge(src, dst, beta, adst, z):
    mesh = plsc.VectorSubcoreMesh(core_axis_name="c", subcore_axis_name="s")
    dbuf = lambda *a: [pltpu.VMEM(*a), pltpu.VMEM(*a)]
    fn = pl.kernel(
        _sc_edge_body,
        compiler_params=pltpu.CompilerParams(use_tc_tiling_on_sc=False),
        out_type=[
            jax.ShapeDtypeStruct((NCORES, NPAD, H * DH), jnp.float32),
            jax.ShapeDtypeStruct((NCORES, NPAD, 16), jnp.float32),
        ],
        mesh=mesh,
        scratch_types=(
            dbuf((CH,), jnp.int32)
            + dbuf((CH,), jnp.int32)
            + dbuf((CH, 16), jnp.float32)
            + dbuf((CH // 8, 128), jnp.float32)
            + dbuf((CH, H * DH + 16), jnp.float32)
            + [
                pltpu.VMEM_SHARED((NPAD, H * DH + 16), jnp.float32),
            ]
            + [pltpu.SemaphoreType.DMA] * 12
        ),
    )
    return fn(src, dst, beta, adst, z)


def _post_body(agg_ref, den_ref, s_ref, e8t_ref, lng_ref, lnb_ref,
               w1_ref, b1_ref, w2_ref, b2_ref, out_ref):
    a = agg_ref[0] + agg_ref[1]
    den = den_ref[0] + den_ref[1]
    den = jnp.where(den > 0.0, den, 1.0)
    den128 = jnp.dot(1.0 / den, e8t_ref[...],
                     preferred_element_type=jnp.float32)
    agg = a * den128
    hfeat = jnp.where(agg > 0.0, agg, jnp.exp(jnp.minimum(agg, 0.0)) - 1.0)
    hfeat = hfeat + s_ref[...]
    mu = jnp.mean(hfeat, axis=-1, keepdims=True)
    xm = hfeat - mu
    var = jnp.mean(xm * xm, axis=-1, keepdims=True)
    xn = xm * lax.rsqrt(var + 1e-6) * lng_ref[...] + lnb_ref[...]
    inter = jax.nn.gelu(jnp.dot(xn, w1_ref[...],
                                preferred_element_type=jnp.float32)
                        + b1_ref[...])
    out_ref[...] = (jnp.dot(inter, w2_ref[...],
                            preferred_element_type=jnp.float32)
                    + b2_ref[...] + hfeat)


def _post(aggraw, denraw, s, e8t, lng, lnb, w1, b1, w2, b2):
    grid = N // BN
    return pl.pallas_call(
        _post_body,
        grid=(grid,),
        in_specs=[
            pl.BlockSpec((NCORES, BN, H * DH), lambda i: (0, i, 0)),
            pl.BlockSpec((NCORES, BN, 16), lambda i: (0, i, 0)),
            pl.BlockSpec((BN, OUT_DIM), lambda i: (i, 0)),
            pl.BlockSpec((16, H * DH), lambda i: (0, 0)),
            pl.BlockSpec((1, OUT_DIM), lambda i: (0, 0)),
            pl.BlockSpec((1, OUT_DIM), lambda i: (0, 0)),
            pl.BlockSpec((OUT_DIM, FFN), lambda i: (0, 0)),
            pl.BlockSpec((1, FFN), lambda i: (0, 0)),
            pl.BlockSpec((FFN, OUT_DIM), lambda i: (0, 0)),
            pl.BlockSpec((1, OUT_DIM), lambda i: (0, 0)),
        ],
        out_specs=pl.BlockSpec((BN, OUT_DIM), lambda i: (i, 0)),
        out_shape=jax.ShapeDtypeStruct((N, OUT_DIM), jnp.float32),
    )(aggraw, denraw, s, e8t, lng, lnb, w1, b1, w2, b2)


def kernel(p, s, edge_attr, Wfc, Wfeat, bfeat, attn_a, W1, b1, W2, b2,
           ln_g, ln_b, edge_index):
    src, dst = _split_edges(edge_index.astype(jnp.int32))

    wfc_flat = Wfc.transpose(1, 0, 2).reshape(IN_DIM, H * DH)
    a1 = attn_a[:, :DH].reshape(1, H * DH)
    a2 = attn_a[:, DH:2 * DH].reshape(1, H * DH)
    a3 = attn_a[:, 2 * DH:]
    w_e16 = jnp.pad(jnp.einsum('hfk,hk->fh', Wfeat, a3), ((0, 0), (0, 8)))
    w_big = jnp.kron(jnp.eye(8, dtype=jnp.float32), w_e16)
    cb_big = jnp.tile(jnp.pad(jnp.einsum('hk,hk->h', bfeat, a3), (0, 8)),
                      8).reshape(1, 128)
    e8 = (jnp.arange(H * DH)[:, None] // DH
          == jnp.arange(16)[None, :]).astype(jnp.float32)

    z144, adst = _node_proj(p, wfc_flat, a1, a2, e8)
    beta = _edge_beta(edge_attr, w_big, cb_big)
    aggraw, denraw = _sc_edge(src, dst, beta, adst, z144)
    e16 = (jnp.arange(16)[:, None]
           == jnp.arange(H * DH)[None, :] // DH).astype(jnp.float32)
    return _post(aggraw, denraw, s, e16, ln_g.reshape(1, OUT_DIM),
                 ln_b.reshape(1, OUT_DIM), W1, b1.reshape(1, FFN),
                 W2, b2.reshape(1, OUT_DIM))


# speedup vs baseline: 87.5613x; 1.0792x over previous
import functools

import jax
import jax.numpy as jnp
from jax import lax
from jax.experimental import pallas as pl
from jax.experimental.pallas import tpu as pltpu
from jax.experimental.pallas import tpu_sc as plsc

N = 10000
E = 320000
IN_DIM = 128
OUT_DIM = 128
H = 8
DH = 16
FEAT = 16
FFN = 512

CH = 80
NCORES = 2
NSUB = 16
NTILES = NCORES * NSUB
NCHUNKS = E // CH
NPAD = 10112
ROWS_PER_TILE = NPAD // NSUB

BN = 2000
BE = 16000


def _node_proj_body(p_ref, wfc_ref, a1_ref, a2_ref, e8_ref,
                    z_ref, adst_ref):
    z = jnp.dot(p_ref[...], wfc_ref[...], preferred_element_type=jnp.float32)
    e8 = e8_ref[...]
    z_ref[:, pl.ds(0, H * DH)] = z
    z_ref[:, pl.ds(H * DH, 16)] = jnp.dot(z * a1_ref[...], e8,
                                          preferred_element_type=jnp.float32)
    adst_ref[...] = jnp.dot(z * a2_ref[...], e8,
                            preferred_element_type=jnp.float32)


def _node_proj(p, wfc_flat, a1, a2, e8):
    grid = N // BN
    return pl.pallas_call(
        _node_proj_body,
        grid=(grid,),
        in_specs=[
            pl.BlockSpec((BN, IN_DIM), lambda i: (i, 0)),
            pl.BlockSpec((IN_DIM, H * DH), lambda i: (0, 0)),
            pl.BlockSpec((1, H * DH), lambda i: (0, 0)),
            pl.BlockSpec((1, H * DH), lambda i: (0, 0)),
            pl.BlockSpec((H * DH, 16), lambda i: (0, 0)),
        ],
        out_specs=[
            pl.BlockSpec((BN, H * DH + 16), lambda i: (i, 0)),
            pl.BlockSpec((BN, 16), lambda i: (i, 0)),
        ],
        out_shape=[
            jax.ShapeDtypeStruct((N, H * DH + 16), jnp.float32),
            jax.ShapeDtypeStruct((N, 16), jnp.float32),
        ],
    )(p, wfc_flat, a1, a2, e8)


E8 = E // 8
BE8 = E8 // 5


def _beta_body(ea_ref, wbig_ref, cb_ref, beta_ref):
    beta_ref[...] = (jnp.dot(ea_ref[...], wbig_ref[...],
                             preferred_element_type=jnp.float32)
                     + cb_ref[...])


def _edge_beta(edge_attr, w_big, cb_big):
    ea_r = edge_attr.reshape(E8, 8 * FEAT)
    return pl.pallas_call(
        _beta_body,
        grid=(E8 // BE8,),
        in_specs=[
            pl.BlockSpec((BE8, 8 * FEAT), lambda i: (i, 0)),
            pl.BlockSpec((8 * FEAT, 128), lambda i: (0, 0)),
            pl.BlockSpec((1, 128), lambda i: (0, 0)),
        ],
        out_specs=pl.BlockSpec((BE8, 128), lambda i: (i, 0)),
        out_shape=jax.ShapeDtypeStruct((E8, 128), jnp.float32),
    )(ea_r, w_big, cb_big)


SPLIT_G = 25
SPLIT_W = E // SPLIT_G
SPLIT_R = SPLIT_W // 128


def _split_edges_body(ei_ref, src_ref, dst_ref):
    for r in range(SPLIT_R):
        src_ref[0, pl.ds(r, 1), :] = ei_ref[pl.ds(0, 1), pl.ds(r * 128, 128)]
        dst_ref[0, pl.ds(r, 1), :] = ei_ref[pl.ds(1, 1), pl.ds(r * 128, 128)]


def _split_edges(edge_index):
    src_pk, dst_pk = pl.pallas_call(
        _split_edges_body,
        grid=(SPLIT_G,),
        in_specs=[pl.BlockSpec((2, SPLIT_W), lambda i: (0, i))],
        out_specs=[
            pl.BlockSpec((1, SPLIT_R, 128), lambda i: (i, 0, 0)),
            pl.BlockSpec((1, SPLIT_R, 128), lambda i: (i, 0, 0)),
        ],
        out_shape=[
            jax.ShapeDtypeStruct((SPLIT_G, SPLIT_R, 128), jnp.int32),
            jax.ShapeDtypeStruct((SPLIT_G, SPLIT_R, 128), jnp.int32),
        ],
    )(edge_index)
    return src_pk.reshape(E), dst_pk.reshape(E)


PER_TILE = NCHUNKS // NTILES


def _sc_edge_body(src_hbm, dst_hbm, beta_hbm, adst_hbm, z_hbm,
                  agg_out, den_out,
                  src_v0, src_v1, dst_v0, dst_v1, dst_v2, avd0, avd1,
                  bv, zr0, zr1, zr2, agg_sh,
                  sgd0, sgd1, sgb, sgz0, sgz1, sgz2,
                  ssa0, ssa1, ssa2, sis0, sis1, sid0, sid1, sid2):
    src_v = (src_v0, src_v1)
    dst_v = (dst_v0, dst_v1, dst_v2)
    avd = (avd0, avd1)
    zr = (zr0, zr1, zr2)
    sgd = (sgd0, sgd1)
    sgz = (sgz0, sgz1, sgz2)
    ssa = (ssa0, ssa1, ssa2)
    sis = (sis0, sis1)
    sid = (sid0, sid1, sid2)

    c = lax.axis_index("c")
    s = lax.axis_index("s")
    wid = c * NSUB + s
    lane = lax.iota(jnp.int32, 16)
    zeros16 = jnp.zeros((16,), jnp.float32)

    def zero_zr(r, carry):
        for q in range(9):
            zr0[r, pl.ds(q * DH, 16)] = zeros16
        return carry
    lax.fori_loop(0, CH, zero_zr, 0)

    base_row = s * ROWS_PER_TILE
    for q in range(ROWS_PER_TILE // CH):
        pltpu.sync_copy(zr0, agg_sh.at[pl.ds(base_row + q * CH, CH)])
    tail = ROWS_PER_TILE % CH
    if tail:
        tbase = base_row + (ROWS_PER_TILE // CH) * CH
        pltpu.sync_copy(zr0.at[pl.ds(0, tail)], agg_sh.at[pl.ds(tbase, tail)])
    plsc.subcore_barrier()

    def issue(k, p2, p3):
        base = (k * NTILES + wid) * CH
        pltpu.async_copy(src_hbm.at[pl.ds(base, CH)], src_v[p2], sis[p2])
        pltpu.async_copy(dst_hbm.at[pl.ds(base, CH)], dst_v[p3], sid[p3])
        pltpu.make_async_copy(src_hbm.at[pl.ds(0, CH)], src_v[p2],
                              sis[p2]).wait()
        pltpu.async_copy(z_hbm.at[src_v[p2]], zr[p3], sgz[p3])
        pltpu.make_async_copy(dst_hbm.at[pl.ds(0, CH)], dst_v[p3],
                              sid[p3]).wait()
        pltpu.async_copy(adst_hbm.at[dst_v[p3]], avd[p2], sgd[p2])

    def issue_bv(k):
        pltpu.async_copy(
            beta_hbm.at[pl.ds((k * NTILES + wid) * (CH // 8), CH // 8)],
            bv, sgb)

    def wait_gathers(p2, p3):
        pltpu.make_async_copy(adst_hbm.at[pl.ds(0, CH)],
                              avd[p2], sgd[p2]).wait()
        pltpu.make_async_copy(beta_hbm.at[pl.ds(0, CH // 8)], bv, sgb).wait()
        pltpu.make_async_copy(z_hbm.at[pl.ds(0, CH)],
                              zr[p3], sgz[p3]).wait()

    def wait_scatter(p3):
        pltpu.make_async_copy(zr[p3], agg_sh.at[pl.ds(0, CH)],
                              ssa[p3]).wait()

    def compute(p2, p3):
        def edge_body(b, carry2):
            t = (zr[p3][b, pl.ds(H * DH, 16)] + avd[p2][b, pl.ds(0, 16)]
                 + bv[b // 8, pl.ds((b % 8) * 16, 16)])
            t = jnp.maximum(t, t * jnp.float32(0.01))
            eev = jnp.exp(t)
            zr[p3][b, pl.ds(H * DH, 16)] = jnp.where(lane < 8, eev, 0.0)
            for h in range(H):
                zr[p3][b, pl.ds(h * DH, 16)] = (
                    zr[p3][b, pl.ds(h * DH, 16)] * eev[h])
            return carry2
        lax.fori_loop(0, CH, edge_body, 0)

    issue(0, 0, 0)
    issue_bv(0)

    def outer(i6, carry):
        for u in range(6):
            k = i6 * 6 + u
            p2, p3 = u % 2, u % 3

            @pl.when(k < PER_TILE)
            def _(k=k, p2=p2, p3=p3):
                @pl.when(k >= 2)
                def _():
                    wait_scatter((p3 + 1) % 3)

                @pl.when(k + 1 < PER_TILE)
                def _():
                    issue(k + 1, (p2 + 1) % 2, (p3 + 1) % 3)
                wait_gathers(p2, p3)
                compute(p2, p3)
                pltpu.async_copy(zr[p3], agg_sh.at[dst_v[p3]], ssa[p3],
                                 add=True)

                @pl.when(k + 1 < PER_TILE)
                def _():
                    issue_bv(k + 1)
        return carry

    lax.fori_loop(0, (PER_TILE + 5) // 6, outer, 0)
    wait_scatter((PER_TILE - 2) % 3)
    wait_scatter((PER_TILE - 1) % 3)
    plsc.subcore_barrier()

    pltpu.sync_copy(
        agg_sh.at[pl.ds(base_row, ROWS_PER_TILE), pl.ds(0, H * DH)],
        agg_out.at[c, pl.ds(base_row, ROWS_PER_TILE)])
    pltpu.sync_copy(
        agg_sh.at[pl.ds(base_row, ROWS_PER_TILE), pl.ds(H * DH, 16)],
        den_out.at[c, pl.ds(base_row, ROWS_PER_TILE)])


def _sc_edge(src, dst, beta, adst, z):
    mesh = plsc.VectorSubcoreMesh(core_axis_name="c", subcore_axis_name="s")
    fn = pl.kernel(
        _sc_edge_body,
        compiler_params=pltpu.CompilerParams(use_tc_tiling_on_sc=False),
        out_type=[
            jax.ShapeDtypeStruct((NCORES, NPAD, H * DH), jnp.float32),
            jax.ShapeDtypeStruct((NCORES, NPAD, 16), jnp.float32),
        ],
        mesh=mesh,
        scratch_types=(
            [pltpu.VMEM((CH,), jnp.int32)] * 2
            + [pltpu.VMEM((CH,), jnp.int32)] * 3
            + [pltpu.VMEM((CH, 16), jnp.float32)] * 2
            + [pltpu.VMEM((CH // 8, 128), jnp.float32)]
            + [pltpu.VMEM((CH, H * DH + 16), jnp.float32)] * 3
            + [
                pltpu.VMEM_SHARED((NPAD, H * DH + 16), jnp.float32),
            ]
            + [pltpu.SemaphoreType.DMA] * 14
        ),
    )
    return fn(src, dst, beta, adst, z)


def _post_body(agg_ref, den_ref, s_ref, e8t_ref, lng_ref, lnb_ref,
               w1_ref, b1_ref, w2_ref, b2_ref, out_ref):
    a = agg_ref[0] + agg_ref[1]
    den = den_ref[0] + den_ref[1]
    den = jnp.where(den > 0.0, den, 1.0)
    den128 = jnp.dot(1.0 / den, e8t_ref[...],
                     preferred_element_type=jnp.float32)
    agg = a * den128
    hfeat = jnp.where(agg > 0.0, agg, jnp.exp(jnp.minimum(agg, 0.0)) - 1.0)
    hfeat = hfeat + s_ref[...]
    mu = jnp.mean(hfeat, axis=-1, keepdims=True)
    xm = hfeat - mu
    var = jnp.mean(xm * xm, axis=-1, keepdims=True)
    xn = xm * lax.rsqrt(var + 1e-6) * lng_ref[...] + lnb_ref[...]
    inter = jax.nn.gelu(jnp.dot(xn, w1_ref[...],
                                preferred_element_type=jnp.float32)
                        + b1_ref[...])
    out_ref[...] = (jnp.dot(inter, w2_ref[...],
                            preferred_element_type=jnp.float32)
                    + b2_ref[...] + hfeat)


def _post(aggraw, denraw, s, e8t, lng, lnb, w1, b1, w2, b2):
    grid = N // BN
    return pl.pallas_call(
        _post_body,
        grid=(grid,),
        in_specs=[
            pl.BlockSpec((NCORES, BN, H * DH), lambda i: (0, i, 0)),
            pl.BlockSpec((NCORES, BN, 16), lambda i: (0, i, 0)),
            pl.BlockSpec((BN, OUT_DIM), lambda i: (i, 0)),
            pl.BlockSpec((16, H * DH), lambda i: (0, 0)),
            pl.BlockSpec((1, OUT_DIM), lambda i: (0, 0)),
            pl.BlockSpec((1, OUT_DIM), lambda i: (0, 0)),
            pl.BlockSpec((OUT_DIM, FFN), lambda i: (0, 0)),
            pl.BlockSpec((1, FFN), lambda i: (0, 0)),
            pl.BlockSpec((FFN, OUT_DIM), lambda i: (0, 0)),
            pl.BlockSpec((1, OUT_DIM), lambda i: (0, 0)),
        ],
        out_specs=pl.BlockSpec((BN, OUT_DIM), lambda i: (i, 0)),
        out_shape=jax.ShapeDtypeStruct((N, OUT_DIM), jnp.float32),
    )(aggraw, denraw, s, e8t, lng, lnb, w1, b1, w2, b2)


def kernel(p, s, edge_attr, Wfc, Wfeat, bfeat, attn_a, W1, b1, W2, b2,
           ln_g, ln_b, edge_index):
    src, dst = _split_edges(edge_index.astype(jnp.int32))

    wfc_flat = Wfc.transpose(1, 0, 2).reshape(IN_DIM, H * DH)
    a1 = attn_a[:, :DH].reshape(1, H * DH)
    a2 = attn_a[:, DH:2 * DH].reshape(1, H * DH)
    a3 = attn_a[:, 2 * DH:]
    w_e16 = jnp.pad(jnp.einsum('hfk,hk->fh', Wfeat, a3), ((0, 0), (0, 8)))
    w_big = jnp.kron(jnp.eye(8, dtype=jnp.float32), w_e16)
    cb_big = jnp.tile(jnp.pad(jnp.einsum('hk,hk->h', bfeat, a3), (0, 8)),
                      8).reshape(1, 128)
    e8 = (jnp.arange(H * DH)[:, None] // DH
          == jnp.arange(16)[None, :]).astype(jnp.float32)

    z144, adst = _node_proj(p, wfc_flat, a1, a2, e8)
    beta = _edge_beta(edge_attr, w_big, cb_big)
    aggraw, denraw = _sc_edge(src, dst, beta, adst, z144)
    e16 = (jnp.arange(16)[:, None]
           == jnp.arange(H * DH)[None, :] // DH).astype(jnp.float32)
    return _post(aggraw, denraw, s, e16, ln_g.reshape(1, OUT_DIM),
                 ln_b.reshape(1, OUT_DIM), W1, b1.reshape(1, FFN),
                 W2, b2.reshape(1, OUT_DIM))


# speedup vs baseline: 87.6068x; 1.0005x over previous
import jax
import jax.numpy as jnp
from jax import lax
from jax.experimental import pallas as pl
from jax.experimental.pallas import tpu as pltpu
from jax.experimental.pallas import tpu_sc as plsc

N = 10000
E = 320000
IN_DIM = 128
OUT_DIM = 128
H = 8
DH = 16
FEAT = 16
FFN = 512

CH = 80
NCORES = 2
NSUB = 16
NTILES = NCORES * NSUB
NCHUNKS = E // CH
NPAD = 10112
ROWS_PER_TILE = NPAD // NSUB

BN = 2000
BE = 16000


def _node_proj_body(p_ref, wfc_ref, a1_ref, a2_ref, e8_ref,
                    z_ref, adst_ref):
    z = jnp.dot(p_ref[...], wfc_ref[...], preferred_element_type=jnp.float32)
    e8 = e8_ref[...]
    z_ref[:, pl.ds(0, H * DH)] = z
    z_ref[:, pl.ds(H * DH, 16)] = jnp.dot(z * a1_ref[...], e8,
                                          preferred_element_type=jnp.float32)
    adst_ref[...] = jnp.dot(z * a2_ref[...], e8,
                            preferred_element_type=jnp.float32)


def _node_proj(p, wfc_flat, a1, a2, e8):
    grid = N // BN
    return pl.pallas_call(
        _node_proj_body,
        grid=(grid,),
        in_specs=[
            pl.BlockSpec((BN, IN_DIM), lambda i: (i, 0)),
            pl.BlockSpec((IN_DIM, H * DH), lambda i: (0, 0)),
            pl.BlockSpec((1, H * DH), lambda i: (0, 0)),
            pl.BlockSpec((1, H * DH), lambda i: (0, 0)),
            pl.BlockSpec((H * DH, 16), lambda i: (0, 0)),
        ],
        out_specs=[
            pl.BlockSpec((BN, H * DH + 16), lambda i: (i, 0)),
            pl.BlockSpec((BN, 16), lambda i: (i, 0)),
        ],
        out_shape=[
            jax.ShapeDtypeStruct((N, H * DH + 16), jnp.float32),
            jax.ShapeDtypeStruct((N, 16), jnp.float32),
        ],
    )(p, wfc_flat, a1, a2, e8)


E8 = E // 8
BE8 = E8 // 5


def _beta_body(ea_ref, wbig_ref, cb_ref, beta_ref):
    beta_ref[...] = (jnp.dot(ea_ref[...], wbig_ref[...],
                             preferred_element_type=jnp.float32)
                     + cb_ref[...])


def _edge_beta(edge_attr, w_big, cb_big):
    ea_r = edge_attr.reshape(E8, 8 * FEAT)
    return pl.pallas_call(
        _beta_body,
        grid=(E8 // BE8,),
        in_specs=[
            pl.BlockSpec((BE8, 8 * FEAT), lambda i: (i, 0)),
            pl.BlockSpec((8 * FEAT, 128), lambda i: (0, 0)),
            pl.BlockSpec((1, 128), lambda i: (0, 0)),
        ],
        out_specs=pl.BlockSpec((BE8, 128), lambda i: (i, 0)),
        out_shape=jax.ShapeDtypeStruct((E8, 128), jnp.float32),
    )(ea_r, w_big, cb_big)


SPLIT_G = 25
SPLIT_W = E // SPLIT_G
SPLIT_R = SPLIT_W // 128


def _split_edges_body(ei_ref, src_ref, dst_ref):
    for r in range(SPLIT_R):
        src_ref[0, pl.ds(r, 1), :] = ei_ref[pl.ds(0, 1), pl.ds(r * 128, 128)]
        dst_ref[0, pl.ds(r, 1), :] = ei_ref[pl.ds(1, 1), pl.ds(r * 128, 128)]


def _split_edges(edge_index):
    src_pk, dst_pk = pl.pallas_call(
        _split_edges_body,
        grid=(SPLIT_G,),
        in_specs=[pl.BlockSpec((2, SPLIT_W), lambda i: (0, i))],
        out_specs=[
            pl.BlockSpec((1, SPLIT_R, 128), lambda i: (i, 0, 0)),
            pl.BlockSpec((1, SPLIT_R, 128), lambda i: (i, 0, 0)),
        ],
        out_shape=[
            jax.ShapeDtypeStruct((SPLIT_G, SPLIT_R, 128), jnp.int32),
            jax.ShapeDtypeStruct((SPLIT_G, SPLIT_R, 128), jnp.int32),
        ],
    )(edge_index)
    return src_pk.reshape(E), dst_pk.reshape(E)


PER_TILE = NCHUNKS // NTILES


def _sc_edge_body(src_hbm, dst_hbm, beta_hbm, adst_hbm, z_hbm,
                  agg_out, den_out,
                  src_v0, src_v1, dst_v0, dst_v1, dst_v2, avd0, avd1,
                  bv, zr0, zr1, zr2, agg_sh,
                  sgd0, sgd1, sgb, sgz0, sgz1, sgz2,
                  ssa0, ssa1, ssa2, sis0, sis1, sid0, sid1, sid2):
    src_v = (src_v0, src_v1)
    dst_v = (dst_v0, dst_v1, dst_v2)
    avd = (avd0, avd1)
    zr = (zr0, zr1, zr2)
    sgd = (sgd0, sgd1)
    sgz = (sgz0, sgz1, sgz2)
    ssa = (ssa0, ssa1, ssa2)
    sis = (sis0, sis1)
    sid = (sid0, sid1, sid2)

    c = lax.axis_index("c")
    s = lax.axis_index("s")
    wid = c * NSUB + s
    lane = lax.iota(jnp.int32, 16)
    zeros16 = jnp.zeros((16,), jnp.float32)

    def zero_zr(r, carry):
        for q in range(9):
            zr0[r, pl.ds(q * DH, 16)] = zeros16
        return carry
    lax.fori_loop(0, CH, zero_zr, 0)

    base_row = s * ROWS_PER_TILE
    for q in range(ROWS_PER_TILE // CH):
        pltpu.sync_copy(zr0, agg_sh.at[pl.ds(base_row + q * CH, CH)])
    tail = ROWS_PER_TILE % CH
    if tail:
        tbase = base_row + (ROWS_PER_TILE // CH) * CH
        pltpu.sync_copy(zr0.at[pl.ds(0, tail)], agg_sh.at[pl.ds(tbase, tail)])
    plsc.subcore_barrier()

    def issue(k, p2, p3):
        base = (k * NTILES + wid) * CH
        pltpu.async_copy(src_hbm.at[pl.ds(base, CH)], src_v[p2], sis[p2])
        pltpu.async_copy(dst_hbm.at[pl.ds(base, CH)], dst_v[p3], sid[p3])
        pltpu.make_async_copy(src_hbm.at[pl.ds(0, CH)], src_v[p2],
                              sis[p2]).wait()
        pltpu.async_copy(z_hbm.at[src_v[p2]], zr[p3], sgz[p3])
        pltpu.make_async_copy(dst_hbm.at[pl.ds(0, CH)], dst_v[p3],
                              sid[p3]).wait()
        pltpu.async_copy(adst_hbm.at[dst_v[p3]], avd[p2], sgd[p2])

    def issue_bv(k):
        pltpu.async_copy(
            beta_hbm.at[pl.ds((k * NTILES + wid) * (CH // 8), CH // 8)],
            bv, sgb)

    def wait_gathers(p2, p3):
        pltpu.make_async_copy(adst_hbm.at[pl.ds(0, CH)],
                              avd[p2], sgd[p2]).wait()
        pltpu.make_async_copy(beta_hbm.at[pl.ds(0, CH // 8)], bv, sgb).wait()
        pltpu.make_async_copy(z_hbm.at[pl.ds(0, CH)],
                              zr[p3], sgz[p3]).wait()

    def wait_scatter(p3):
        pltpu.make_async_copy(zr[p3], agg_sh.at[pl.ds(0, CH)],
                              ssa[p3]).wait()

    def compute(p2, p3):
        def edge_body(b, carry2):
            t = (zr[p3][b, pl.ds(H * DH, 16)] + avd[p2][b, pl.ds(0, 16)]
                 + bv[b // 8, pl.ds((b % 8) * 16, 16)])
            t = jnp.maximum(t, t * jnp.float32(0.01))
            eev = jnp.exp(t)
            zr[p3][b, pl.ds(H * DH, 16)] = jnp.where(lane < 8, eev, 0.0)
            for h in range(H):
                zr[p3][b, pl.ds(h * DH, 16)] = (
                    zr[p3][b, pl.ds(h * DH, 16)] * eev[h])
            return carry2
        lax.fori_loop(0, CH, edge_body, 0)

    issue(0, 0, 0)
    issue_bv(0)

    def outer(i6, carry):
        for u in range(6):
            k = i6 * 6 + u
            p2, p3 = u % 2, u % 3

            @pl.when(k < PER_TILE)
            def _(k=k, p2=p2, p3=p3):
                @pl.when(k >= 2)
                def _():
                    wait_scatter((p3 + 1) % 3)

                @pl.when(k + 1 < PER_TILE)
                def _():
                    issue(k + 1, (p2 + 1) % 2, (p3 + 1) % 3)
                wait_gathers(p2, p3)
                compute(p2, p3)
                pltpu.async_copy(zr[p3], agg_sh.at[dst_v[p3]], ssa[p3],
                                 add=True)

                @pl.when(k + 1 < PER_TILE)
                def _():
                    issue_bv(k + 1)
        return carry

    lax.fori_loop(0, (PER_TILE + 5) // 6, outer, 0)
    wait_scatter((PER_TILE - 2) % 3)
    wait_scatter((PER_TILE - 1) % 3)
    plsc.subcore_barrier()

    pltpu.sync_copy(
        agg_sh.at[pl.ds(base_row, ROWS_PER_TILE), pl.ds(0, H * DH)],
        agg_out.at[c, pl.ds(base_row, ROWS_PER_TILE)])
    pltpu.sync_copy(
        agg_sh.at[pl.ds(base_row, ROWS_PER_TILE), pl.ds(H * DH, 16)],
        den_out.at[c, pl.ds(base_row, ROWS_PER_TILE)])


def _sc_edge(src, dst, beta, adst, z):
    mesh = plsc.VectorSubcoreMesh(core_axis_name="c", subcore_axis_name="s")
    fn = pl.kernel(
        _sc_edge_body,
        compiler_params=pltpu.CompilerParams(use_tc_tiling_on_sc=False),
        out_type=[
            jax.ShapeDtypeStruct((NCORES, NPAD, H * DH), jnp.float32),
            jax.ShapeDtypeStruct((NCORES, NPAD, 16), jnp.float32),
        ],
        mesh=mesh,
        scratch_types=(
            [pltpu.VMEM((CH,), jnp.int32)] * 2
            + [pltpu.VMEM((CH,), jnp.int32)] * 3
            + [pltpu.VMEM((CH, 16), jnp.float32)] * 2
            + [pltpu.VMEM((CH // 8, 128), jnp.float32)]
            + [pltpu.VMEM((CH, H * DH + 16), jnp.float32)] * 3
            + [
                pltpu.VMEM_SHARED((NPAD, H * DH + 16), jnp.float32),
            ]
            + [pltpu.SemaphoreType.DMA] * 14
        ),
    )
    return fn(src, dst, beta, adst, z)


def _post_body(agg_ref, den_ref, s_ref, e8t_ref, lng_ref, lnb_ref,
               w1_ref, b1_ref, w2_ref, b2_ref, out_ref):
    a = agg_ref[0] + agg_ref[1]
    den = den_ref[0] + den_ref[1]
    den = jnp.where(den > 0.0, den, 1.0)
    den128 = jnp.dot(1.0 / den, e8t_ref[...],
                     preferred_element_type=jnp.float32)
    agg = a * den128
    hfeat = jnp.where(agg > 0.0, agg, jnp.exp(jnp.minimum(agg, 0.0)) - 1.0)
    hfeat = hfeat + s_ref[...]
    mu = jnp.mean(hfeat, axis=-1, keepdims=True)
    xm = hfeat - mu
    var = jnp.mean(xm * xm, axis=-1, keepdims=True)
    xn = xm * lax.rsqrt(var + 1e-6) * lng_ref[...] + lnb_ref[...]
    inter = jax.nn.gelu(jnp.dot(xn, w1_ref[...],
                                preferred_element_type=jnp.float32)
                        + b1_ref[...])
    out_ref[...] = (jnp.dot(inter, w2_ref[...],
                            preferred_element_type=jnp.float32)
                    + b2_ref[...] + hfeat)


def _post(aggraw, denraw, s, e8t, lng, lnb, w1, b1, w2, b2):
    grid = N // BN
    return pl.pallas_call(
        _post_body,
        grid=(grid,),
        in_specs=[
            pl.BlockSpec((NCORES, BN, H * DH), lambda i: (0, i, 0)),
            pl.BlockSpec((NCORES, BN, 16), lambda i: (0, i, 0)),
            pl.BlockSpec((BN, OUT_DIM), lambda i: (i, 0)),
            pl.BlockSpec((16, H * DH), lambda i: (0, 0)),
            pl.BlockSpec((1, OUT_DIM), lambda i: (0, 0)),
            pl.BlockSpec((1, OUT_DIM), lambda i: (0, 0)),
            pl.BlockSpec((OUT_DIM, FFN), lambda i: (0, 0)),
            pl.BlockSpec((1, FFN), lambda i: (0, 0)),
            pl.BlockSpec((FFN, OUT_DIM), lambda i: (0, 0)),
            pl.BlockSpec((1, OUT_DIM), lambda i: (0, 0)),
        ],
        out_specs=pl.BlockSpec((BN, OUT_DIM), lambda i: (i, 0)),
        out_shape=jax.ShapeDtypeStruct((N, OUT_DIM), jnp.float32),
    )(aggraw, denraw, s, e8t, lng, lnb, w1, b1, w2, b2)


def kernel(p, s, edge_attr, Wfc, Wfeat, bfeat, attn_a, W1, b1, W2, b2,
           ln_g, ln_b, edge_index):
    src, dst = _split_edges(edge_index.astype(jnp.int32))

    wfc_flat = Wfc.transpose(1, 0, 2).reshape(IN_DIM, H * DH)
    a1 = attn_a[:, :DH].reshape(1, H * DH)
    a2 = attn_a[:, DH:2 * DH].reshape(1, H * DH)
    a3 = attn_a[:, 2 * DH:]
    w_e16 = jnp.pad(jnp.einsum('hfk,hk->fh', Wfeat, a3), ((0, 0), (0, 8)))
    w_big = jnp.kron(jnp.eye(8, dtype=jnp.float32), w_e16)
    cb_big = jnp.tile(jnp.pad(jnp.einsum('hk,hk->h', bfeat, a3), (0, 8)),
                      8).reshape(1, 128)
    e8 = (jnp.arange(H * DH)[:, None] // DH
          == jnp.arange(16)[None, :]).astype(jnp.float32)

    z144, adst = _node_proj(p, wfc_flat, a1, a2, e8)
    beta = _edge_beta(edge_attr, w_big, cb_big)
    aggraw, denraw = _sc_edge(src, dst, beta, adst, z144)
    e16 = (jnp.arange(16)[:, None]
           == jnp.arange(H * DH)[None, :] // DH).astype(jnp.float32)
    return _post(aggraw, denraw, s, e16, ln_g.reshape(1, OUT_DIM),
                 ln_b.reshape(1, OUT_DIM), W1, b1.reshape(1, FFN),
                 W2, b2.reshape(1, OUT_DIM))
